# baseline (speedup 1.0000x reference)
import functools

import jax
import jax.numpy as jnp
from jax import lax
from jax.experimental import pallas as pl
from jax.experimental.pallas import tpu as pltpu
from jax.experimental.pallas import tpu_sc as plsc

D = 1024
E = 8
K = 2
D_FF = 256
N = 4096

T_BLK = 256
P_BLK = 256
NT = N // T_BLK
NB = 40
MAXP = NB * P_BLK
NEG_INF = -1e30


def _k1_body(x_ref, rw_ref, e_ref, rank_ref, wb_ref, counts_ref, xb_ref, carry):
    t = pl.program_id(0)

    @pl.when(t == 0)
    def _():
        carry[...] = jnp.zeros_like(carry)

    x = x_ref[...]
    logits = lax.dot_general(x, rw_ref[...], (((1,), (1,)), ((), ())),
                             preferred_element_type=jnp.float32)
    e_iota = lax.broadcasted_iota(jnp.int32, logits.shape, 1)
    m1 = jnp.max(logits, axis=1, keepdims=True)
    i1 = jnp.min(jnp.where(logits == m1, e_iota, E), axis=1, keepdims=True)
    masked = jnp.where(e_iota == i1, NEG_INF, logits)
    m2 = jnp.max(masked, axis=1, keepdims=True)
    i2 = jnp.min(jnp.where(masked == m2, e_iota, E), axis=1, keepdims=True)
    u = jnp.exp(m2 - m1)
    w2 = u / (1.0 + u)
    w1 = 1.0 - w2

    oh0 = (e_iota == i1).astype(jnp.float32)
    oh1 = (e_iota == i2).astype(jnp.float32)
    oh01 = oh0 + oh1
    r_iota = lax.broadcasted_iota(jnp.int32, (T_BLK, T_BLK), 0)
    c_iota = lax.broadcasted_iota(jnp.int32, (T_BLK, T_BLK), 1)
    lstrict = (r_iota > c_iota).astype(jnp.float32)
    within = lax.dot_general(lstrict, oh01, (((1,), (0,)), ((), ())),
                             preferred_element_type=jnp.float32)
    cum0 = carry[...] + within
    cum1 = cum0 + oh0
    rank0 = jnp.sum(cum0 * oh0, axis=1, keepdims=True)
    rank1 = jnp.sum(cum1 * oh1, axis=1, keepdims=True)
    carry[...] += jnp.sum(oh01, axis=0, keepdims=True)

    e_ref[0] = i1
    e_ref[1] = i2
    rank_ref[0] = rank0.astype(jnp.int32)
    rank_ref[1] = rank1.astype(jnp.int32)
    wb_ref[0] = jnp.broadcast_to(w1, (T_BLK, 16))
    wb_ref[1] = jnp.broadcast_to(w2, (T_BLK, 16))
    counts_ref[...] = carry[...]
    xb_ref[...] = x.astype(jnp.bfloat16)


def _k1(x, route_W):
    return pl.pallas_call(
        _k1_body,
        grid=(NT,),
        in_specs=[
            pl.BlockSpec((T_BLK, D), lambda t: (t, 0)),
            pl.BlockSpec((E, D), lambda t: (0, 0)),
        ],
        out_specs=[
            pl.BlockSpec((K, T_BLK, 1), lambda t: (0, t, 0)),
            pl.BlockSpec((K, T_BLK, 1), lambda t: (0, t, 0)),
            pl.BlockSpec((K, T_BLK, 16), lambda t: (0, t, 0)),
            pl.BlockSpec((1, E), lambda t: (0, 0)),
            pl.BlockSpec((T_BLK, D), lambda t: (t, 0)),
        ],
        out_shape=[
            jax.ShapeDtypeStruct((K, N, 1), jnp.int32),
            jax.ShapeDtypeStruct((K, N, 1), jnp.int32),
            jax.ShapeDtypeStruct((K, N, 16), jnp.float32),
            jax.ShapeDtypeStruct((1, E), jnp.float32),
            jax.ShapeDtypeStruct((N, D), jnp.bfloat16),
        ],
        scratch_shapes=[pltpu.VMEM((1, E), jnp.float32)],
    )(x, route_W)


def _k2_body(counts_ref, e_ref, rank_ref, pos_ref, be_ref):
    counts = counts_ref[...].astype(jnp.int32)
    nblk = (counts + (P_BLK - 1)) >> 8
    r8 = lax.broadcasted_iota(jnp.int32, (E, E), 0)
    c8 = lax.broadcasted_iota(jnp.int32, (E, E), 1)
    u_excl = (r8 < c8).astype(jnp.float32)
    u_incl = (r8 <= c8).astype(jnp.float32)
    nblk_f = nblk.astype(jnp.float32)
    off_blocks = lax.dot_general(nblk_f, u_excl, (((1,), (0,)), ((), ())),
                                 preferred_element_type=jnp.float32)
    cum_incl = lax.dot_general(nblk_f, u_incl, (((1,), (0,)), ((), ())),
                               preferred_element_type=jnp.float32)
    padded_off = off_blocks * float(P_BLK)

    e_blk = e_ref[0]
    rank = rank_ref[0]
    lane8 = lax.broadcasted_iota(jnp.int32, (T_BLK, E), 1)
    oh = (e_blk == lane8).astype(jnp.float32)
    base = jnp.sum(oh * padded_off, axis=1, keepdims=True)
    pos_ref[0] = base.astype(jnp.int32) + rank

    b_iota = lax.broadcasted_iota(jnp.int32, (64, E), 0).astype(jnp.float32)
    be = jnp.sum((b_iota >= cum_incl).astype(jnp.float32),
                 axis=1, keepdims=True).astype(jnp.int32)
    be_ref[...] = jnp.minimum(be, E - 1)


def _k2(counts, e_arr, rank_arr):
    return pl.pallas_call(
        _k2_body,
        grid=(K, NT),
        in_specs=[
            pl.BlockSpec((1, E), lambda k, t: (0, 0)),
            pl.BlockSpec((1, T_BLK, 1), lambda k, t: (k, t, 0)),
            pl.BlockSpec((1, T_BLK, 1), lambda k, t: (k, t, 0)),
        ],
        out_specs=[
            pl.BlockSpec((1, T_BLK, 1), lambda k, t: (k, t, 0)),
            pl.BlockSpec((64, 1), lambda k, t: (0, 0)),
        ],
        out_shape=[
            jax.ShapeDtypeStruct((K, N, 1), jnp.int32),
            jax.ShapeDtypeStruct((64, 1), jnp.int32),
        ],
    )(counts, e_arr, rank_arr)


def _k4_body(be_ref, xs_ref, w1_ref, b1_ref, w2_ref, b2_ref, y_ref):
    xs = xs_ref[...].astype(jnp.float32)
    h = lax.dot_general(xs, w1_ref[0], (((1,), (1,)), ((), ())),
                        preferred_element_type=jnp.float32)
    h = jnp.maximum(h + b1_ref[0], 0.0)
    y = lax.dot_general(h, w2_ref[0], (((1,), (1,)), ((), ())),
                        preferred_element_type=jnp.float32)
    y_ref[...] = jnp.maximum(y + b2_ref[0], 0.0)


def _k4(be, x_sorted, W1, b1, W2, b2):
    grid_spec = pltpu.PrefetchScalarGridSpec(
        num_scalar_prefetch=1,
        grid=(NB,),
        in_specs=[
            pl.BlockSpec((P_BLK, D), lambda i, be_ref: (i, 0)),
            pl.BlockSpec((1, D_FF, D), lambda i, be_ref: (be_ref[i], 0, 0)),
            pl.BlockSpec((1, 1, D_FF), lambda i, be_ref: (be_ref[i], 0, 0)),
            pl.BlockSpec((1, D, D_FF), lambda i, be_ref: (be_ref[i], 0, 0)),
            pl.BlockSpec((1, 1, D), lambda i, be_ref: (be_ref[i], 0, 0)),
        ],
        out_specs=pl.BlockSpec((P_BLK, D), lambda i, be_ref: (i, 0)),
    )
    return pl.pallas_call(
        _k4_body,
        grid_spec=grid_spec,
        out_shape=jax.ShapeDtypeStruct((MAXP, D), jnp.float32),
    )(be, x_sorted, W1, b1.reshape(E, 1, D_FF), W2, b2.reshape(E, 1, D))


NW = 32
TOK_W = N // NW
C3 = 64
C5 = 16

_SC_MESH = dict(core_axis_name="c", subcore_axis_name="s")


@functools.partial(
    pl.kernel,
    mesh=plsc.VectorSubcoreMesh(**_SC_MESH),
    out_type=jax.ShapeDtypeStruct((MAXP, D // 2), jnp.int32),
    scratch_types=[
        pltpu.VMEM((C3, D // 2), jnp.int32),
        pltpu.VMEM((C3,), jnp.int32),
        pltpu.VMEM((C3,), jnp.int32),
        pltpu.SemaphoreType.DMA,
        pltpu.SemaphoreType.DMA,
    ],
)
def _k3(x_hbm, p0_hbm, p1_hbm, xs_hbm, xv, i0v, i1v, sem0, sem1):
    wid = lax.axis_index("s") * 2 + lax.axis_index("c")
    for c in range(TOK_W // C3):
        base = wid * TOK_W + c * C3
        pltpu.sync_copy(x_hbm.at[pl.ds(base, C3)], xv)
        pltpu.sync_copy(p0_hbm.at[pl.ds(base, C3)], i0v)
        pltpu.sync_copy(p1_hbm.at[pl.ds(base, C3)], i1v)
        a = pltpu.async_copy(xv, xs_hbm.at[i0v], sem0)
        b = pltpu.async_copy(xv, xs_hbm.at[i1v], sem1)
        a.wait()
        b.wait()


_NC5 = TOK_W // C5


@functools.partial(
    pl.kernel,
    mesh=plsc.VectorSubcoreMesh(**_SC_MESH),
    out_type=jax.ShapeDtypeStruct((N, D), jnp.float32),
    scratch_types=[
        pltpu.VMEM((TOK_W,), jnp.int32),
        pltpu.VMEM((TOK_W,), jnp.int32),
        pltpu.VMEM((TOK_W, 16), jnp.float32),
        pltpu.VMEM((TOK_W, 16), jnp.float32),
        pltpu.VMEM((2, C5, D), jnp.float32),
        pltpu.VMEM((2, C5, D), jnp.float32),
        pltpu.SemaphoreType.DMA,
        pltpu.SemaphoreType.DMA,
        pltpu.SemaphoreType.DMA,
        pltpu.SemaphoreType.DMA,
        pltpu.SemaphoreType.DMA,
        pltpu.SemaphoreType.DMA,
    ],
)
def _k5(y_hbm, p0_hbm, p1_hbm, w0_hbm, w1_hbm, out_hbm,
        i0v, i1v, w0v, w1v, y0v, y1v,
        g0a, g0b, g1a, g1b, s0, s1):
    wid = lax.axis_index("s") * 2 + lax.axis_index("c")
    base = wid * TOK_W
    pltpu.sync_copy(p0_hbm.at[pl.ds(base, TOK_W)], i0v)
    pltpu.sync_copy(p1_hbm.at[pl.ds(base, TOK_W)], i1v)
    pltpu.sync_copy(w0_hbm.at[pl.ds(base, TOK_W)], w0v)
    pltpu.sync_copy(w1_hbm.at[pl.ds(base, TOK_W)], w1v)
    gsems = ((g0a, g1a), (g0b, g1b))
    ssems = (s0, s1)
    gather_pend = [None, None]
    store_pend = [None, None]

    def issue_gathers(c):
        buf = c % 2
        ga, gb = gsems[buf]
        idx0 = i0v[pl.ds(c * C5, C5)]
        idx1 = i1v[pl.ds(c * C5, C5)]
        a = pltpu.async_copy(y_hbm.at[idx0], y0v.at[buf], ga)
        b = pltpu.async_copy(y_hbm.at[idx1], y1v.at[buf], gb)
        gather_pend[buf] = (a, b)

    issue_gathers(0)
    for c in range(_NC5):
        buf = c % 2
        a, b = gather_pend[buf]
        a.wait()
        b.wait()
        if c + 1 < _NC5:
            nbuf = (c + 1) % 2
            if store_pend[nbuf] is not None:
                store_pend[nbuf].wait()
            issue_gathers(c + 1)

        def body(t, _, c=c, buf=buf):
            w0s = w0v[c * C5 + t, :]
            w1s = w1v[c * C5 + t, :]
            for j in range(D // 16):
                sl = pl.ds(j * 16, 16)
                y0v[buf, t, sl] = w0s * y0v[buf, t, sl] + w1s * y1v[buf, t, sl]
            return 0

        lax.fori_loop(0, C5, body, 0)
        store_pend[buf] = pltpu.async_copy(
            y0v.at[buf], out_hbm.at[pl.ds(base + c * C5, C5)], ssems[buf])
    for sp in store_pend:
        sp.wait()


def kernel(x, route_W, W1, b1, W2, b2):
    e_arr, rank_arr, wb, counts, xb = _k1(x, route_W)
    pos, be = _k2(counts, e_arr, rank_arr)
    pos0 = pos[0, :, 0]
    pos1 = pos[1, :, 0]
    be1d = be[:, 0]
    xb32 = lax.bitcast_convert_type(xb.reshape(N, D // 2, 2), jnp.int32)
    xs32 = _k3(xb32, pos0, pos1)
    x_sorted = lax.bitcast_convert_type(xs32, jnp.bfloat16).reshape(MAXP, D)
    y_sorted = _k4(be1d, x_sorted, W1, b1, W2, b2)
    return _k5(y_sorted, pos0, pos1, wb[0], wb[1])


# speedup vs baseline: 2.7166x; 2.7166x over previous
import functools

import jax
import jax.numpy as jnp
from jax import lax
from jax.experimental import pallas as pl
from jax.experimental.pallas import tpu as pltpu
from jax.experimental.pallas import tpu_sc as plsc

D = 1024
E = 8
K = 2
D_FF = 256
N = 4096

D2 = D // 2
T_BLK = 256
P_BLK = 256
NT = N // T_BLK
NB = 40
MAXP = NB * P_BLK
NEG_INF = -1e30


def _k1_body(x_ref, rw_ref, e_ref, rank_ref, wb_ref, counts_ref, xb_ref, carry):
    t = pl.program_id(0)

    @pl.when(t == 0)
    def _():
        carry[...] = jnp.zeros_like(carry)

    x = x_ref[...]
    logits = lax.dot_general(x, rw_ref[...], (((1,), (1,)), ((), ())),
                             preferred_element_type=jnp.float32)
    e_iota = lax.broadcasted_iota(jnp.int32, logits.shape, 1)
    m1 = jnp.max(logits, axis=1, keepdims=True)
    i1 = jnp.min(jnp.where(logits == m1, e_iota, E), axis=1, keepdims=True)
    masked = jnp.where(e_iota == i1, NEG_INF, logits)
    m2 = jnp.max(masked, axis=1, keepdims=True)
    i2 = jnp.min(jnp.where(masked == m2, e_iota, E), axis=1, keepdims=True)
    u = jnp.exp(m2 - m1)
    w2 = u / (1.0 + u)
    w1 = 1.0 - w2

    oh0 = (e_iota == i1).astype(jnp.float32)
    oh1 = (e_iota == i2).astype(jnp.float32)
    oh01 = oh0 + oh1
    r_iota = lax.broadcasted_iota(jnp.int32, (T_BLK, T_BLK), 0)
    c_iota = lax.broadcasted_iota(jnp.int32, (T_BLK, T_BLK), 1)
    lstrict = (r_iota > c_iota).astype(jnp.float32)
    within = lax.dot_general(lstrict, oh01, (((1,), (0,)), ((), ())),
                             preferred_element_type=jnp.float32)
    cum0 = carry[...] + within
    cum1 = cum0 + oh0
    rank0 = jnp.sum(cum0 * oh0, axis=1, keepdims=True)
    rank1 = jnp.sum(cum1 * oh1, axis=1, keepdims=True)
    carry[...] += jnp.sum(oh01, axis=0, keepdims=True)

    e_ref[0] = i1
    e_ref[1] = i2
    rank_ref[0] = rank0.astype(jnp.int32)
    rank_ref[1] = rank1.astype(jnp.int32)
    wb_ref[0] = jnp.broadcast_to(w1, (T_BLK, 16))
    wb_ref[1] = jnp.broadcast_to(w2, (T_BLK, 16))
    counts_ref[...] = carry[...]
    xb = x.astype(jnp.bfloat16)
    ul = lax.bitcast_convert_type(xb[:, :D2], jnp.uint16).astype(jnp.int32)
    uh = lax.bitcast_convert_type(xb[:, D2:], jnp.uint16).astype(jnp.int32)
    xb_ref[...] = ul | (uh << 16)


def _k1(x, route_W):
    return pl.pallas_call(
        _k1_body,
        grid=(NT,),
        in_specs=[
            pl.BlockSpec((T_BLK, D), lambda t: (t, 0)),
            pl.BlockSpec((E, D), lambda t: (0, 0)),
        ],
        out_specs=[
            pl.BlockSpec((K, T_BLK, 1), lambda t: (0, t, 0)),
            pl.BlockSpec((K, T_BLK, 1), lambda t: (0, t, 0)),
            pl.BlockSpec((K, T_BLK, 16), lambda t: (0, t, 0)),
            pl.BlockSpec((1, E), lambda t: (0, 0)),
            pl.BlockSpec((T_BLK, D2), lambda t: (t, 0)),
        ],
        out_shape=[
            jax.ShapeDtypeStruct((K, N, 1), jnp.int32),
            jax.ShapeDtypeStruct((K, N, 1), jnp.int32),
            jax.ShapeDtypeStruct((K, N, 16), jnp.float32),
            jax.ShapeDtypeStruct((1, E), jnp.float32),
            jax.ShapeDtypeStruct((N, D2), jnp.int32),
        ],
        scratch_shapes=[pltpu.VMEM((1, E), jnp.float32)],
    )(x, route_W)


def _k2_body(counts_ref, e_ref, rank_ref, pos_ref, be_ref):
    counts = counts_ref[...].astype(jnp.int32)
    nblk = (counts + (P_BLK - 1)) >> 8
    r8 = lax.broadcasted_iota(jnp.int32, (E, E), 0)
    c8 = lax.broadcasted_iota(jnp.int32, (E, E), 1)
    u_excl = (r8 < c8).astype(jnp.float32)
    u_incl = (r8 <= c8).astype(jnp.float32)
    nblk_f = nblk.astype(jnp.float32)
    off_blocks = lax.dot_general(nblk_f, u_excl, (((1,), (0,)), ((), ())),
                                 preferred_element_type=jnp.float32)
    cum_incl = lax.dot_general(nblk_f, u_incl, (((1,), (0,)), ((), ())),
                               preferred_element_type=jnp.float32)
    padded_off = off_blocks * float(P_BLK)

    e_blk = e_ref[0]
    rank = rank_ref[0]
    lane8 = lax.broadcasted_iota(jnp.int32, (T_BLK, E), 1)
    oh = (e_blk == lane8).astype(jnp.float32)
    base = jnp.sum(oh * padded_off, axis=1, keepdims=True)
    pos_ref[0] = base.astype(jnp.int32) + rank

    b_iota = lax.broadcasted_iota(jnp.int32, (64, E), 0).astype(jnp.float32)
    be = jnp.sum((b_iota >= cum_incl).astype(jnp.float32),
                 axis=1, keepdims=True).astype(jnp.int32)
    be_ref[...] = jnp.minimum(be, E - 1)


def _k2(counts, e_arr, rank_arr):
    return pl.pallas_call(
        _k2_body,
        grid=(K, NT),
        in_specs=[
            pl.BlockSpec((1, E), lambda k, t: (0, 0)),
            pl.BlockSpec((1, T_BLK, 1), lambda k, t: (k, t, 0)),
            pl.BlockSpec((1, T_BLK, 1), lambda k, t: (k, t, 0)),
        ],
        out_specs=[
            pl.BlockSpec((1, T_BLK, 1), lambda k, t: (k, t, 0)),
            pl.BlockSpec((64, 1), lambda k, t: (0, 0)),
        ],
        out_shape=[
            jax.ShapeDtypeStruct((K, N, 1), jnp.int32),
            jax.ShapeDtypeStruct((64, 1), jnp.int32),
        ],
    )(counts, e_arr, rank_arr)


def _unpack_bf16(u):
    lo = lax.bitcast_convert_type((u & 0xFFFF).astype(jnp.uint16), jnp.bfloat16)
    hi = lax.bitcast_convert_type((u >> 16).astype(jnp.uint16), jnp.bfloat16)
    return lo, hi


def _pack_bf16(lo, hi):
    ul = lax.bitcast_convert_type(lo, jnp.uint16).astype(jnp.int32)
    uh = lax.bitcast_convert_type(hi, jnp.uint16).astype(jnp.int32)
    return ul | (uh << 16)


def _k4_body(be_ref, xs_ref, w1_ref, b1_ref, w2_ref, b2_ref, y_ref):
    xlo, xhi = _unpack_bf16(xs_ref[...])
    w1b = w1_ref[0].astype(jnp.bfloat16)
    h = lax.dot_general(xlo, w1b[:, :D2], (((1,), (1,)), ((), ())),
                        preferred_element_type=jnp.float32)
    h += lax.dot_general(xhi, w1b[:, D2:], (((1,), (1,)), ((), ())),
                         preferred_element_type=jnp.float32)
    h = jnp.maximum(h + b1_ref[0], 0.0).astype(jnp.bfloat16)
    w2b = w2_ref[0].astype(jnp.bfloat16)
    y = lax.dot_general(h, w2b, (((1,), (1,)), ((), ())),
                        preferred_element_type=jnp.float32)
    y = jnp.maximum(y + b2_ref[0], 0.0).astype(jnp.bfloat16)
    y_ref[...] = _pack_bf16(y[:, :D2], y[:, D2:])


def _k4(be, x_sorted, W1, b1, W2, b2):
    grid_spec = pltpu.PrefetchScalarGridSpec(
        num_scalar_prefetch=1,
        grid=(NB,),
        in_specs=[
            pl.BlockSpec((P_BLK, D2), lambda i, be_ref: (i, 0)),
            pl.BlockSpec((1, D_FF, D), lambda i, be_ref: (be_ref[i], 0, 0)),
            pl.BlockSpec((1, 1, D_FF), lambda i, be_ref: (be_ref[i], 0, 0)),
            pl.BlockSpec((1, D, D_FF), lambda i, be_ref: (be_ref[i], 0, 0)),
            pl.BlockSpec((1, 1, D), lambda i, be_ref: (be_ref[i], 0, 0)),
        ],
        out_specs=pl.BlockSpec((P_BLK, D2), lambda i, be_ref: (i, 0)),
    )
    return pl.pallas_call(
        _k4_body,
        grid_spec=grid_spec,
        out_shape=jax.ShapeDtypeStruct((MAXP, D2), jnp.int32),
    )(be, x_sorted, W1, b1.reshape(E, 1, D_FF), W2, b2.reshape(E, 1, D))


NW = 32
TOK_W = N // NW
C3 = 64
C5 = 16

_SC_MESH = dict(core_axis_name="c", subcore_axis_name="s")


@functools.partial(
    pl.kernel,
    mesh=plsc.VectorSubcoreMesh(**_SC_MESH),
    out_type=jax.ShapeDtypeStruct((MAXP, D // 2), jnp.int32),
    scratch_types=[
        pltpu.VMEM((C3, D // 2), jnp.int32),
        pltpu.VMEM((C3,), jnp.int32),
        pltpu.VMEM((C3,), jnp.int32),
        pltpu.SemaphoreType.DMA,
        pltpu.SemaphoreType.DMA,
    ],
)
def _k3(x_hbm, p0_hbm, p1_hbm, xs_hbm, xv, i0v, i1v, sem0, sem1):
    wid = lax.axis_index("s") * 2 + lax.axis_index("c")
    for c in range(TOK_W // C3):
        base = wid * TOK_W + c * C3
        pltpu.sync_copy(x_hbm.at[pl.ds(base, C3)], xv)
        pltpu.sync_copy(p0_hbm.at[pl.ds(base, C3)], i0v)
        pltpu.sync_copy(p1_hbm.at[pl.ds(base, C3)], i1v)
        a = pltpu.async_copy(xv, xs_hbm.at[i0v], sem0)
        b = pltpu.async_copy(xv, xs_hbm.at[i1v], sem1)
        a.wait()
        b.wait()


_NC5 = TOK_W // C5


@functools.partial(
    pl.kernel,
    mesh=plsc.VectorSubcoreMesh(**_SC_MESH),
    out_type=jax.ShapeDtypeStruct((N, D), jnp.float32),
    scratch_types=[
        pltpu.VMEM((TOK_W,), jnp.int32),
        pltpu.VMEM((TOK_W,), jnp.int32),
        pltpu.VMEM((TOK_W, 16), jnp.float32),
        pltpu.VMEM((TOK_W, 16), jnp.float32),
        pltpu.VMEM((2, C5, D2), jnp.int32),
        pltpu.VMEM((2, C5, D2), jnp.int32),
        pltpu.VMEM((2, C5, D), jnp.float32),
        pltpu.SemaphoreType.DMA,
        pltpu.SemaphoreType.DMA,
        pltpu.SemaphoreType.DMA,
        pltpu.SemaphoreType.DMA,
        pltpu.SemaphoreType.DMA,
        pltpu.SemaphoreType.DMA,
    ],
)
def _k5(y_hbm, p0_hbm, p1_hbm, w0_hbm, w1_hbm, out_hbm,
        i0v, i1v, w0v, w1v, y0v, y1v, ov,
        g0a, g0b, g1a, g1b, s0, s1):
    wid = lax.axis_index("s") * 2 + lax.axis_index("c")
    base = wid * TOK_W
    pltpu.sync_copy(p0_hbm.at[pl.ds(base, TOK_W)], i0v)
    pltpu.sync_copy(p1_hbm.at[pl.ds(base, TOK_W)], i1v)
    pltpu.sync_copy(w0_hbm.at[pl.ds(base, TOK_W)], w0v)
    pltpu.sync_copy(w1_hbm.at[pl.ds(base, TOK_W)], w1v)
    gsems = ((g0a, g1a), (g0b, g1b))
    ssems = (s0, s1)
    gather_pend = [None, None]
    store_pend = [None, None]

    def issue_gathers(c):
        buf = c % 2
        ga, gb = gsems[buf]
        idx0 = i0v[pl.ds(c * C5, C5)]
        idx1 = i1v[pl.ds(c * C5, C5)]
        a = pltpu.async_copy(y_hbm.at[idx0], y0v.at[buf], ga)
        b = pltpu.async_copy(y_hbm.at[idx1], y1v.at[buf], gb)
        gather_pend[buf] = (a, b)

    issue_gathers(0)
    for c in range(_NC5):
        buf = c % 2
        a, b = gather_pend[buf]
        a.wait()
        b.wait()
        if c + 1 < _NC5:
            nbuf = (c + 1) % 2
            if store_pend[nbuf] is not None:
                store_pend[nbuf].wait()
            issue_gathers(c + 1)

        def body(t, _, c=c, buf=buf):
            w0s = w0v[c * C5 + t, :]
            w1s = w1v[c * C5 + t, :]
            for j in range(D2 // 16):
                sl = pl.ds(j * 16, 16)
                slh = pl.ds(D2 + j * 16, 16)
                u0 = y0v[buf, t, sl]
                u1 = y1v[buf, t, sl]
                lo0 = lax.bitcast_convert_type(u0 << 16, jnp.float32)
                lo1 = lax.bitcast_convert_type(u1 << 16, jnp.float32)
                hi0 = lax.bitcast_convert_type(u0 & jnp.int32(-65536), jnp.float32)
                hi1 = lax.bitcast_convert_type(u1 & jnp.int32(-65536), jnp.float32)
                ov[buf, t, sl] = w0s * lo0 + w1s * lo1
                ov[buf, t, slh] = w0s * hi0 + w1s * hi1
            return 0

        lax.fori_loop(0, C5, body, 0)
        store_pend[buf] = pltpu.async_copy(
            ov.at[buf], out_hbm.at[pl.ds(base + c * C5, C5)], ssems[buf])
    for sp in store_pend:
        sp.wait()


def kernel(x, route_W, W1, b1, W2, b2):
    e_arr, rank_arr, wb, counts, xb = _k1(x, route_W)
    pos, be = _k2(counts, e_arr, rank_arr)
    pos0 = pos[0, :, 0]
    pos1 = pos[1, :, 0]
    be1d = be[:, 0]
    x_sorted = _k3(xb, pos0, pos1)
    y_sorted = _k4(be1d, x_sorted, W1, b1, W2, b2)
    return _k5(y_sorted, pos0, pos1, wb[0], wb[1])


# speedup vs baseline: 2.7366x; 1.0073x over previous
import functools

import jax
import jax.numpy as jnp
from jax import lax
from jax.experimental import pallas as pl
from jax.experimental.pallas import tpu as pltpu
from jax.experimental.pallas import tpu_sc as plsc

D = 1024
E = 8
K = 2
D_FF = 256
N = 4096

D2 = D // 2
T_BLK = 256
P_BLK = 256
NT = N // T_BLK
NB = 40
MAXP = NB * P_BLK
NEG_INF = -1e30


def _k1_body(x_ref, rw_ref, e_ref, rank_ref, wb_ref, counts_ref, xb_ref, carry):
    t = pl.program_id(0)

    @pl.when(t == 0)
    def _():
        carry[...] = jnp.zeros_like(carry)

    x = x_ref[...]
    logits = lax.dot_general(x, rw_ref[...], (((1,), (1,)), ((), ())),
                             preferred_element_type=jnp.float32)
    e_iota = lax.broadcasted_iota(jnp.int32, logits.shape, 1)
    m1 = jnp.max(logits, axis=1, keepdims=True)
    i1 = jnp.min(jnp.where(logits == m1, e_iota, E), axis=1, keepdims=True)
    masked = jnp.where(e_iota == i1, NEG_INF, logits)
    m2 = jnp.max(masked, axis=1, keepdims=True)
    i2 = jnp.min(jnp.where(masked == m2, e_iota, E), axis=1, keepdims=True)
    u = jnp.exp(m2 - m1)
    w2 = u / (1.0 + u)
    w1 = 1.0 - w2

    oh0 = (e_iota == i1).astype(jnp.float32)
    oh1 = (e_iota == i2).astype(jnp.float32)
    oh01 = oh0 + oh1
    r_iota = lax.broadcasted_iota(jnp.int32, (T_BLK, T_BLK), 0)
    c_iota = lax.broadcasted_iota(jnp.int32, (T_BLK, T_BLK), 1)
    lstrict = (r_iota > c_iota).astype(jnp.float32)
    within = lax.dot_general(lstrict, oh01, (((1,), (0,)), ((), ())),
                             preferred_element_type=jnp.float32)
    cum0 = carry[...] + within
    cum1 = cum0 + oh0
    rank0 = jnp.sum(cum0 * oh0, axis=1, keepdims=True)
    rank1 = jnp.sum(cum1 * oh1, axis=1, keepdims=True)
    carry[...] += jnp.sum(oh01, axis=0, keepdims=True)

    e_ref[0] = i1
    e_ref[1] = i2
    rank_ref[0] = rank0.astype(jnp.int32)
    rank_ref[1] = rank1.astype(jnp.int32)
    wb_ref[0] = jnp.broadcast_to(w1, (T_BLK, 16))
    wb_ref[1] = jnp.broadcast_to(w2, (T_BLK, 16))
    counts_ref[...] = carry[...]
    xb = x.astype(jnp.bfloat16)
    ul = lax.bitcast_convert_type(xb[:, :D2], jnp.uint16).astype(jnp.int32)
    uh = lax.bitcast_convert_type(xb[:, D2:], jnp.uint16).astype(jnp.int32)
    xb_ref[...] = ul | (uh << 16)


def _k1(x, route_W):
    return pl.pallas_call(
        _k1_body,
        grid=(NT,),
        in_specs=[
            pl.BlockSpec((T_BLK, D), lambda t: (t, 0)),
            pl.BlockSpec((E, D), lambda t: (0, 0)),
        ],
        out_specs=[
            pl.BlockSpec((K, T_BLK, 1), lambda t: (0, t, 0)),
            pl.BlockSpec((K, T_BLK, 1), lambda t: (0, t, 0)),
            pl.BlockSpec((K, T_BLK, 16), lambda t: (0, t, 0)),
            pl.BlockSpec((1, E), lambda t: (0, 0)),
            pl.BlockSpec((T_BLK, D2), lambda t: (t, 0)),
        ],
        out_shape=[
            jax.ShapeDtypeStruct((K, N, 1), jnp.int32),
            jax.ShapeDtypeStruct((K, N, 1), jnp.int32),
            jax.ShapeDtypeStruct((K, N, 16), jnp.float32),
            jax.ShapeDtypeStruct((1, E), jnp.float32),
            jax.ShapeDtypeStruct((N, D2), jnp.int32),
        ],
        scratch_shapes=[pltpu.VMEM((1, E), jnp.float32)],
    )(x, route_W)


def _k2_body(counts_ref, e_ref, rank_ref, pos_ref, be_ref):
    counts = counts_ref[...].astype(jnp.int32)
    nblk = (counts + (P_BLK - 1)) >> 8
    r8 = lax.broadcasted_iota(jnp.int32, (E, E), 0)
    c8 = lax.broadcasted_iota(jnp.int32, (E, E), 1)
    u_excl = (r8 < c8).astype(jnp.float32)
    u_incl = (r8 <= c8).astype(jnp.float32)
    nblk_f = nblk.astype(jnp.float32)
    off_blocks = lax.dot_general(nblk_f, u_excl, (((1,), (0,)), ((), ())),
                                 preferred_element_type=jnp.float32)
    cum_incl = lax.dot_general(nblk_f, u_incl, (((1,), (0,)), ((), ())),
                               preferred_element_type=jnp.float32)
    padded_off = off_blocks * float(P_BLK)

    e_blk = e_ref[0]
    rank = rank_ref[0]
    lane8 = lax.broadcasted_iota(jnp.int32, (T_BLK, E), 1)
    oh = (e_blk == lane8).astype(jnp.float32)
    base = jnp.sum(oh * padded_off, axis=1, keepdims=True)
    pos_ref[0] = base.astype(jnp.int32) + rank

    b_iota = lax.broadcasted_iota(jnp.int32, (64, E), 0).astype(jnp.float32)
    be = jnp.sum((b_iota >= cum_incl).astype(jnp.float32),
                 axis=1, keepdims=True).astype(jnp.int32)
    be_ref[...] = jnp.minimum(be, E - 1)


def _k2(counts, e_arr, rank_arr):
    return pl.pallas_call(
        _k2_body,
        grid=(K, NT),
        in_specs=[
            pl.BlockSpec((1, E), lambda k, t: (0, 0)),
            pl.BlockSpec((1, T_BLK, 1), lambda k, t: (k, t, 0)),
            pl.BlockSpec((1, T_BLK, 1), lambda k, t: (k, t, 0)),
        ],
        out_specs=[
            pl.BlockSpec((1, T_BLK, 1), lambda k, t: (k, t, 0)),
            pl.BlockSpec((64, 1), lambda k, t: (0, 0)),
        ],
        out_shape=[
            jax.ShapeDtypeStruct((K, N, 1), jnp.int32),
            jax.ShapeDtypeStruct((64, 1), jnp.int32),
        ],
    )(counts, e_arr, rank_arr)


def _unpack_bf16(u):
    lo = lax.bitcast_convert_type((u & 0xFFFF).astype(jnp.uint16), jnp.bfloat16)
    hi = lax.bitcast_convert_type((u >> 16).astype(jnp.uint16), jnp.bfloat16)
    return lo, hi


def _pack_bf16(lo, hi):
    ul = lax.bitcast_convert_type(lo, jnp.uint16).astype(jnp.int32)
    uh = lax.bitcast_convert_type(hi, jnp.uint16).astype(jnp.int32)
    return ul | (uh << 16)


def _k4_body(be_ref, xs_ref, w1_ref, b1_ref, w2_ref, b2_ref, y_ref):
    xlo, xhi = _unpack_bf16(xs_ref[...])
    w1b = w1_ref[0].astype(jnp.bfloat16)
    h = lax.dot_general(xlo, w1b[:, :D2], (((1,), (1,)), ((), ())),
                        preferred_element_type=jnp.float32)
    h += lax.dot_general(xhi, w1b[:, D2:], (((1,), (1,)), ((), ())),
                         preferred_element_type=jnp.float32)
    h = jnp.maximum(h + b1_ref[0], 0.0).astype(jnp.bfloat16)
    w2b = w2_ref[0].astype(jnp.bfloat16)
    y = lax.dot_general(h, w2b, (((1,), (1,)), ((), ())),
                        preferred_element_type=jnp.float32)
    y = jnp.maximum(y + b2_ref[0], 0.0).astype(jnp.bfloat16)
    y_ref[...] = _pack_bf16(y[:, :D2], y[:, D2:])


def _k4(be, x_sorted, W1, b1, W2, b2):
    grid_spec = pltpu.PrefetchScalarGridSpec(
        num_scalar_prefetch=1,
        grid=(NB,),
        in_specs=[
            pl.BlockSpec((P_BLK, D2), lambda i, be_ref: (i, 0)),
            pl.BlockSpec((1, D_FF, D), lambda i, be_ref: (be_ref[i], 0, 0)),
            pl.BlockSpec((1, 1, D_FF), lambda i, be_ref: (be_ref[i], 0, 0)),
            pl.BlockSpec((1, D, D_FF), lambda i, be_ref: (be_ref[i], 0, 0)),
            pl.BlockSpec((1, 1, D), lambda i, be_ref: (be_ref[i], 0, 0)),
        ],
        out_specs=pl.BlockSpec((P_BLK, D2), lambda i, be_ref: (i, 0)),
    )
    return pl.pallas_call(
        _k4_body,
        grid_spec=grid_spec,
        out_shape=jax.ShapeDtypeStruct((MAXP, D2), jnp.int32),
    )(be, x_sorted, W1, b1.reshape(E, 1, D_FF), W2, b2.reshape(E, 1, D))


NW = 32
TOK_W = N // NW
C3 = 64
C5 = 16

_SC_MESH = dict(core_axis_name="c", subcore_axis_name="s")


@functools.partial(
    pl.kernel,
    mesh=plsc.VectorSubcoreMesh(**_SC_MESH),
    out_type=jax.ShapeDtypeStruct((MAXP, D // 2), jnp.int32),
    scratch_types=[
        pltpu.VMEM((C3, D // 2), jnp.int32),
        pltpu.VMEM((C3,), jnp.int32),
        pltpu.VMEM((C3,), jnp.int32),
        pltpu.SemaphoreType.DMA,
        pltpu.SemaphoreType.DMA,
    ],
)
def _k3(x_hbm, p_hbm, xs_hbm, xv, i0v, i1v, sem0, sem1):
    wid = lax.axis_index("s") * 2 + lax.axis_index("c")
    for c in range(TOK_W // C3):
        base = wid * TOK_W + c * C3
        pltpu.sync_copy(x_hbm.at[pl.ds(base, C3)], xv)
        pltpu.sync_copy(p_hbm.at[pl.ds(base, C3)], i0v)
        pltpu.sync_copy(p_hbm.at[pl.ds(N + base, C3)], i1v)
        a = pltpu.async_copy(xv, xs_hbm.at[i0v], sem0)
        b = pltpu.async_copy(xv, xs_hbm.at[i1v], sem1)
        a.wait()
        b.wait()


_NC5 = TOK_W // C5


@functools.partial(
    pl.kernel,
    mesh=plsc.VectorSubcoreMesh(**_SC_MESH),
    out_type=jax.ShapeDtypeStruct((N, D), jnp.float32),
    scratch_types=[
        pltpu.VMEM((TOK_W,), jnp.int32),
        pltpu.VMEM((TOK_W,), jnp.int32),
        pltpu.VMEM((TOK_W, 16), jnp.float32),
        pltpu.VMEM((TOK_W, 16), jnp.float32),
        pltpu.VMEM((2, C5, D2), jnp.int32),
        pltpu.VMEM((2, C5, D2), jnp.int32),
        pltpu.VMEM((2, C5, D), jnp.float32),
        pltpu.SemaphoreType.DMA,
        pltpu.SemaphoreType.DMA,
        pltpu.SemaphoreType.DMA,
        pltpu.SemaphoreType.DMA,
        pltpu.SemaphoreType.DMA,
        pltpu.SemaphoreType.DMA,
    ],
)
def _k5(y_hbm, p_hbm, w_hbm, out_hbm,
        i0v, i1v, w0v, w1v, y0v, y1v, ov,
        g0a, g0b, g1a, g1b, s0, s1):
    wid = lax.axis_index("s") * 2 + lax.axis_index("c")
    base = wid * TOK_W
    pltpu.sync_copy(p_hbm.at[pl.ds(base, TOK_W)], i0v)
    pltpu.sync_copy(p_hbm.at[pl.ds(N + base, TOK_W)], i1v)
    pltpu.sync_copy(w_hbm.at[pl.ds(base, TOK_W)], w0v)
    pltpu.sync_copy(w_hbm.at[pl.ds(N + base, TOK_W)], w1v)
    gsems = ((g0a, g1a), (g0b, g1b))
    ssems = (s0, s1)
    gather_pend = [None, None]
    store_pend = [None, None]

    def issue_gathers(c):
        buf = c % 2
        ga, gb = gsems[buf]
        idx0 = i0v[pl.ds(c * C5, C5)]
        idx1 = i1v[pl.ds(c * C5, C5)]
        a = pltpu.async_copy(y_hbm.at[idx0], y0v.at[buf], ga)
        b = pltpu.async_copy(y_hbm.at[idx1], y1v.at[buf], gb)
        gather_pend[buf] = (a, b)

    issue_gathers(0)
    for c in range(_NC5):
        buf = c % 2
        a, b = gather_pend[buf]
        a.wait()
        b.wait()
        if c + 1 < _NC5:
            nbuf = (c + 1) % 2
            if store_pend[nbuf] is not None:
                store_pend[nbuf].wait()
            issue_gathers(c + 1)

        def body(t, _, c=c, buf=buf):
            w0s = w0v[c * C5 + t, :]
            w1s = w1v[c * C5 + t, :]
            for j in range(D2 // 16):
                sl = pl.ds(j * 16, 16)
                slh = pl.ds(D2 + j * 16, 16)
                u0 = y0v[buf, t, sl]
                u1 = y1v[buf, t, sl]
                lo0 = lax.bitcast_convert_type(u0 << 16, jnp.float32)
                lo1 = lax.bitcast_convert_type(u1 << 16, jnp.float32)
                hi0 = lax.bitcast_convert_type(u0 & jnp.int32(-65536), jnp.float32)
                hi1 = lax.bitcast_convert_type(u1 & jnp.int32(-65536), jnp.float32)
                ov[buf, t, sl] = w0s * lo0 + w1s * lo1
                ov[buf, t, slh] = w0s * hi0 + w1s * hi1
            return 0

        lax.fori_loop(0, C5, body, 0)
        store_pend[buf] = pltpu.async_copy(
            ov.at[buf], out_hbm.at[pl.ds(base + c * C5, C5)], ssems[buf])
    for sp in store_pend:
        sp.wait()


def kernel(x, route_W, W1, b1, W2, b2):
    e_arr, rank_arr, wb, counts, xb = _k1(x, route_W)
    pos, be = _k2(counts, e_arr, rank_arr)
    p_flat = pos.reshape(K * N)
    wb_flat = wb.reshape(K * N, 16)
    x_sorted = _k3(xb, p_flat)
    y_sorted = _k4(be.reshape(64), x_sorted, W1, b1, W2, b2)
    return _k5(y_sorted, p_flat, wb_flat)


# speedup vs baseline: 2.7807x; 1.0161x over previous
import functools

import jax
import jax.numpy as jnp
from jax import lax
from jax.experimental import pallas as pl
from jax.experimental.pallas import tpu as pltpu
from jax.experimental.pallas import tpu_sc as plsc

D = 1024
E = 8
K = 2
D_FF = 256
N = 4096

D2 = D // 2
T_BLK = 256
P_BLK = 256
NT = N // T_BLK
NB = 40
MAXP = NB * P_BLK
NEG_INF = -1e30


def _k1_body(x_ref, rw_ref, e_ref, rank_ref, wb_ref, counts_ref, xb_ref, carry):
    t = pl.program_id(0)

    @pl.when(t == 0)
    def _():
        carry[...] = jnp.zeros_like(carry)

    x = x_ref[...]
    logits = lax.dot_general(x, rw_ref[...], (((1,), (1,)), ((), ())),
                             preferred_element_type=jnp.float32)
    e_iota = lax.broadcasted_iota(jnp.int32, logits.shape, 1)
    m1 = jnp.max(logits, axis=1, keepdims=True)
    i1 = jnp.min(jnp.where(logits == m1, e_iota, E), axis=1, keepdims=True)
    masked = jnp.where(e_iota == i1, NEG_INF, logits)
    m2 = jnp.max(masked, axis=1, keepdims=True)
    i2 = jnp.min(jnp.where(masked == m2, e_iota, E), axis=1, keepdims=True)
    u = jnp.exp(m2 - m1)
    w2 = u / (1.0 + u)
    w1 = 1.0 - w2

    oh0 = (e_iota == i1).astype(jnp.float32)
    oh1 = (e_iota == i2).astype(jnp.float32)
    oh01 = oh0 + oh1
    r_iota = lax.broadcasted_iota(jnp.int32, (T_BLK, T_BLK), 0)
    c_iota = lax.broadcasted_iota(jnp.int32, (T_BLK, T_BLK), 1)
    lstrict = (r_iota > c_iota).astype(jnp.float32)
    within = lax.dot_general(lstrict, oh01, (((1,), (0,)), ((), ())),
                             preferred_element_type=jnp.float32)
    cum0 = carry[...] + within
    cum1 = cum0 + oh0
    rank0 = jnp.sum(cum0 * oh0, axis=1, keepdims=True)
    rank1 = jnp.sum(cum1 * oh1, axis=1, keepdims=True)
    carry[...] += jnp.sum(oh01, axis=0, keepdims=True)

    e_ref[0] = i1
    e_ref[1] = i2
    rank_ref[0] = rank0.astype(jnp.int32)
    rank_ref[1] = rank1.astype(jnp.int32)
    wb_ref[0] = jnp.broadcast_to(w1, (T_BLK, 16))
    wb_ref[1] = jnp.broadcast_to(w2, (T_BLK, 16))
    counts_ref[...] = carry[...]
    xb = x.astype(jnp.bfloat16)
    ul = lax.bitcast_convert_type(xb[:, :D2], jnp.uint16).astype(jnp.int32)
    uh = lax.bitcast_convert_type(xb[:, D2:], jnp.uint16).astype(jnp.int32)
    xb_ref[...] = ul | (uh << 16)


def _k1(x, route_W):
    return pl.pallas_call(
        _k1_body,
        grid=(NT,),
        in_specs=[
            pl.BlockSpec((T_BLK, D), lambda t: (t, 0)),
            pl.BlockSpec((E, D), lambda t: (0, 0)),
        ],
        out_specs=[
            pl.BlockSpec((K, T_BLK, 1), lambda t: (0, t, 0)),
            pl.BlockSpec((K, T_BLK, 1), lambda t: (0, t, 0)),
            pl.BlockSpec((K, T_BLK, 16), lambda t: (0, t, 0)),
            pl.BlockSpec((1, E), lambda t: (0, 0)),
            pl.BlockSpec((T_BLK, D2), lambda t: (t, 0)),
        ],
        out_shape=[
            jax.ShapeDtypeStruct((K, N, 1), jnp.int32),
            jax.ShapeDtypeStruct((K, N, 1), jnp.int32),
            jax.ShapeDtypeStruct((K, N, 16), jnp.float32),
            jax.ShapeDtypeStruct((1, E), jnp.float32),
            jax.ShapeDtypeStruct((N, D2), jnp.int32),
        ],
        scratch_shapes=[pltpu.VMEM((1, E), jnp.float32)],
    )(x, route_W)


def _k2_body(counts_ref, e_ref, rank_ref, pos_ref, be_ref):
    counts = counts_ref[...].astype(jnp.int32)
    nblk = (counts + (P_BLK - 1)) >> 8
    r8 = lax.broadcasted_iota(jnp.int32, (E, E), 0)
    c8 = lax.broadcasted_iota(jnp.int32, (E, E), 1)
    u_excl = (r8 < c8).astype(jnp.float32)
    u_incl = (r8 <= c8).astype(jnp.float32)
    nblk_f = nblk.astype(jnp.float32)
    off_blocks = lax.dot_general(nblk_f, u_excl, (((1,), (0,)), ((), ())),
                                 preferred_element_type=jnp.float32)
    cum_incl = lax.dot_general(nblk_f, u_incl, (((1,), (0,)), ((), ())),
                               preferred_element_type=jnp.float32)
    padded_off = off_blocks * float(P_BLK)

    e_blk = e_ref[0]
    rank = rank_ref[0]
    lane8 = lax.broadcasted_iota(jnp.int32, (T_BLK, E), 1)
    oh = (e_blk == lane8).astype(jnp.float32)
    base = jnp.sum(oh * padded_off, axis=1, keepdims=True)
    pos_ref[0] = base.astype(jnp.int32) + rank

    b_iota = lax.broadcasted_iota(jnp.int32, (64, E), 0).astype(jnp.float32)
    be = jnp.sum((b_iota >= cum_incl).astype(jnp.float32),
                 axis=1, keepdims=True).astype(jnp.int32)
    be_ref[...] = jnp.minimum(be, E - 1)


def _k2(counts, e_arr, rank_arr):
    return pl.pallas_call(
        _k2_body,
        grid=(K, NT),
        in_specs=[
            pl.BlockSpec((1, E), lambda k, t: (0, 0)),
            pl.BlockSpec((1, T_BLK, 1), lambda k, t: (k, t, 0)),
            pl.BlockSpec((1, T_BLK, 1), lambda k, t: (k, t, 0)),
        ],
        out_specs=[
            pl.BlockSpec((1, T_BLK, 1), lambda k, t: (k, t, 0)),
            pl.BlockSpec((64, 1), lambda k, t: (0, 0)),
        ],
        out_shape=[
            jax.ShapeDtypeStruct((K, N, 1), jnp.int32),
            jax.ShapeDtypeStruct((64, 1), jnp.int32),
        ],
    )(counts, e_arr, rank_arr)


def _unpack_bf16(u):
    lo = lax.bitcast_convert_type((u & 0xFFFF).astype(jnp.uint16), jnp.bfloat16)
    hi = lax.bitcast_convert_type((u >> 16).astype(jnp.uint16), jnp.bfloat16)
    return lo, hi


def _pack_bf16(lo, hi):
    ul = lax.bitcast_convert_type(lo, jnp.uint16).astype(jnp.int32)
    uh = lax.bitcast_convert_type(hi, jnp.uint16).astype(jnp.int32)
    return ul | (uh << 16)


def _k4_body(be_ref, xs_ref, w1_ref, b1_ref, w2_ref, b2_ref, y_ref):
    xlo, xhi = _unpack_bf16(xs_ref[...])
    w1b = w1_ref[0].astype(jnp.bfloat16)
    h = lax.dot_general(xlo, w1b[:, :D2], (((1,), (1,)), ((), ())),
                        preferred_element_type=jnp.float32)
    h += lax.dot_general(xhi, w1b[:, D2:], (((1,), (1,)), ((), ())),
                         preferred_element_type=jnp.float32)
    h = jnp.maximum(h + b1_ref[0], 0.0).astype(jnp.bfloat16)
    w2b = w2_ref[0].astype(jnp.bfloat16)
    y = lax.dot_general(h, w2b, (((1,), (1,)), ((), ())),
                        preferred_element_type=jnp.float32)
    y = jnp.maximum(y + b2_ref[0], 0.0).astype(jnp.bfloat16)
    y_ref[...] = _pack_bf16(y[:, :D2], y[:, D2:])


def _k4(be, x_sorted, W1, b1, W2, b2):
    grid_spec = pltpu.PrefetchScalarGridSpec(
        num_scalar_prefetch=1,
        grid=(NB,),
        in_specs=[
            pl.BlockSpec((P_BLK, D2), lambda i, be_ref: (i, 0)),
            pl.BlockSpec((1, D_FF, D), lambda i, be_ref: (be_ref[i], 0, 0)),
            pl.BlockSpec((1, 1, D_FF), lambda i, be_ref: (be_ref[i], 0, 0)),
            pl.BlockSpec((1, D, D_FF), lambda i, be_ref: (be_ref[i], 0, 0)),
            pl.BlockSpec((1, 1, D), lambda i, be_ref: (be_ref[i], 0, 0)),
        ],
        out_specs=pl.BlockSpec((P_BLK, D2), lambda i, be_ref: (i, 0)),
    )
    return pl.pallas_call(
        _k4_body,
        grid_spec=grid_spec,
        out_shape=jax.ShapeDtypeStruct((MAXP, D2), jnp.int32),
    )(be, x_sorted, W1, b1.reshape(E, 1, D_FF), W2, b2.reshape(E, 1, D))


NW = 32
TOK_W = N // NW
C3 = 64
C5 = 16

_SC_MESH = dict(core_axis_name="c", subcore_axis_name="s")


@functools.partial(
    pl.kernel,
    mesh=plsc.VectorSubcoreMesh(**_SC_MESH),
    out_type=jax.ShapeDtypeStruct((MAXP, D // 2), jnp.int32),
    scratch_types=[
        pltpu.VMEM((C3, D // 2), jnp.int32),
        pltpu.VMEM((C3,), jnp.int32),
        pltpu.VMEM((C3,), jnp.int32),
        pltpu.SemaphoreType.DMA,
        pltpu.SemaphoreType.DMA,
    ],
)
def _k3(x_hbm, p_hbm, xs_hbm, xv, i0v, i1v, sem0, sem1):
    wid = lax.axis_index("s") * 2 + lax.axis_index("c")
    for c in range(TOK_W // C3):
        base = wid * TOK_W + c * C3
        pltpu.sync_copy(x_hbm.at[pl.ds(base, C3)], xv)
        pltpu.sync_copy(p_hbm.at[pl.ds(base, C3)], i0v)
        pltpu.sync_copy(p_hbm.at[pl.ds(N + base, C3)], i1v)
        a = pltpu.async_copy(xv, xs_hbm.at[i0v], sem0)
        b = pltpu.async_copy(xv, xs_hbm.at[i1v], sem1)
        a.wait()
        b.wait()


_NC5 = TOK_W // C5


@functools.partial(
    pl.kernel,
    mesh=plsc.VectorSubcoreMesh(**_SC_MESH),
    out_type=jax.ShapeDtypeStruct((N, D), jnp.float32),
    scratch_types=[
        pltpu.VMEM((TOK_W,), jnp.int32),
        pltpu.VMEM((TOK_W,), jnp.int32),
        pltpu.VMEM((TOK_W, 16), jnp.float32),
        pltpu.VMEM((TOK_W, 16), jnp.float32),
        pltpu.VMEM((3, C5, D2), jnp.int32),
        pltpu.VMEM((3, C5, D2), jnp.int32),
        pltpu.VMEM((2, C5, D), jnp.float32),
        pltpu.SemaphoreType.DMA,
        pltpu.SemaphoreType.DMA,
        pltpu.SemaphoreType.DMA,
        pltpu.SemaphoreType.DMA,
        pltpu.SemaphoreType.DMA,
        pltpu.SemaphoreType.DMA,
        pltpu.SemaphoreType.DMA,
        pltpu.SemaphoreType.DMA,
    ],
)
def _k5(y_hbm, p_hbm, w_hbm, out_hbm,
        i0v, i1v, w0v, w1v, y0v, y1v, ov,
        g0a, g0b, g0c, g1a, g1b, g1c, s0, s1):
    wid = lax.axis_index("s") * 2 + lax.axis_index("c")
    base = wid * TOK_W
    pltpu.sync_copy(p_hbm.at[pl.ds(base, TOK_W)], i0v)
    pltpu.sync_copy(p_hbm.at[pl.ds(N + base, TOK_W)], i1v)
    pltpu.sync_copy(w_hbm.at[pl.ds(base, TOK_W)], w0v)
    pltpu.sync_copy(w_hbm.at[pl.ds(N + base, TOK_W)], w1v)
    gsems = ((g0a, g1a), (g0b, g1b), (g0c, g1c))
    ssems = (s0, s1)
    gather_pend = [None, None, None]
    store_pend = [None, None]

    def issue_gathers(c):
        buf = c % 3
        ga, gb = gsems[buf]
        idx0 = i0v[pl.ds(c * C5, C5)]
        idx1 = i1v[pl.ds(c * C5, C5)]
        a = pltpu.async_copy(y_hbm.at[idx0], y0v.at[buf], ga)
        b = pltpu.async_copy(y_hbm.at[idx1], y1v.at[buf], gb)
        gather_pend[buf] = (a, b)

    for c in range(2):
        issue_gathers(c)
    for c in range(_NC5):
        buf = c % 3
        a, b = gather_pend[buf]
        a.wait()
        b.wait()
        if c + 2 < _NC5:
            issue_gathers(c + 2)
        obuf = c % 2
        if store_pend[obuf] is not None:
            store_pend[obuf].wait()

        def body(t, _, c=c, buf=buf, obuf=obuf):
            w0s = w0v[c * C5 + t, :]
            w1s = w1v[c * C5 + t, :]
            for j in range(D2 // 16):
                sl = pl.ds(j * 16, 16)
                slh = pl.ds(D2 + j * 16, 16)
                u0 = y0v[buf, t, sl]
                u1 = y1v[buf, t, sl]
                lo0 = lax.bitcast_convert_type(u0 << 16, jnp.float32)
                lo1 = lax.bitcast_convert_type(u1 << 16, jnp.float32)
                hi0 = lax.bitcast_convert_type(u0 & jnp.int32(-65536), jnp.float32)
                hi1 = lax.bitcast_convert_type(u1 & jnp.int32(-65536), jnp.float32)
                ov[obuf, t, sl] = w0s * lo0 + w1s * lo1
                ov[obuf, t, slh] = w0s * hi0 + w1s * hi1
            return 0

        lax.fori_loop(0, C5, body, 0)
        store_pend[obuf] = pltpu.async_copy(
            ov.at[obuf], out_hbm.at[pl.ds(base + c * C5, C5)], ssems[obuf])
    for sp in store_pend:
        sp.wait()


def kernel(x, route_W, W1, b1, W2, b2):
    e_arr, rank_arr, wb, counts, xb = _k1(x, route_W)
    pos, be = _k2(counts, e_arr, rank_arr)
    p_flat = pos.reshape(K * N)
    wb_flat = wb.reshape(K * N, 16)
    x_sorted = _k3(xb, p_flat)
    y_sorted = _k4(be.reshape(64), x_sorted, W1, b1, W2, b2)
    return _k5(y_sorted, p_flat, wb_flat)


# speedup vs baseline: 3.0456x; 1.0952x over previous
import functools

import jax
import jax.numpy as jnp
from jax import lax
from jax.experimental import pallas as pl
from jax.experimental.pallas import tpu as pltpu
from jax.experimental.pallas import tpu_sc as plsc

D = 1024
E = 8
K = 2
D_FF = 256
N = 4096

D2 = D // 2
T_BLK = 512
P_BLK = 256
NT = N // T_BLK
NB = 40
MAXP = NB * P_BLK
NEG_INF = -1e30


def _k1_body(x_ref, rw_ref, e_ref, rank_ref, wb_ref, counts_ref, xb_ref, carry):
    t = pl.program_id(0)

    @pl.when(t == 0)
    def _():
        carry[...] = jnp.zeros_like(carry)

    x = x_ref[...]
    logits = lax.dot_general(x, rw_ref[...], (((1,), (1,)), ((), ())),
                             preferred_element_type=jnp.float32)
    e_iota = lax.broadcasted_iota(jnp.int32, logits.shape, 1)
    m1 = jnp.max(logits, axis=1, keepdims=True)
    i1 = jnp.min(jnp.where(logits == m1, e_iota, E), axis=1, keepdims=True)
    masked = jnp.where(e_iota == i1, NEG_INF, logits)
    m2 = jnp.max(masked, axis=1, keepdims=True)
    i2 = jnp.min(jnp.where(masked == m2, e_iota, E), axis=1, keepdims=True)
    u = jnp.exp(m2 - m1)
    w2 = u / (1.0 + u)
    w1 = 1.0 - w2

    oh0 = (e_iota == i1).astype(jnp.float32)
    oh1 = (e_iota == i2).astype(jnp.float32)
    oh01 = oh0 + oh1
    r_iota = lax.broadcasted_iota(jnp.int32, (T_BLK, T_BLK), 0)
    c_iota = lax.broadcasted_iota(jnp.int32, (T_BLK, T_BLK), 1)
    lstrict = (r_iota > c_iota).astype(jnp.float32)
    within = lax.dot_general(lstrict, oh01, (((1,), (0,)), ((), ())),
                             preferred_element_type=jnp.float32)
    cum0 = carry[...] + within
    cum1 = cum0 + oh0
    rank0 = jnp.sum(cum0 * oh0, axis=1, keepdims=True)
    rank1 = jnp.sum(cum1 * oh1, axis=1, keepdims=True)
    carry[...] += jnp.sum(oh01, axis=0, keepdims=True)

    e_ref[0] = i1
    e_ref[1] = i2
    rank_ref[0] = rank0.astype(jnp.int32)
    rank_ref[1] = rank1.astype(jnp.int32)
    wb_ref[0] = jnp.broadcast_to(w1, (T_BLK, 16))
    wb_ref[1] = jnp.broadcast_to(w2, (T_BLK, 16))
    counts_ref[...] = carry[...]
    xb = x.astype(jnp.bfloat16)
    ul = lax.bitcast_convert_type(xb[:, :D2], jnp.uint16).astype(jnp.int32)
    uh = lax.bitcast_convert_type(xb[:, D2:], jnp.uint16).astype(jnp.int32)
    xb_ref[...] = ul | (uh << 16)


def _k1(x, route_W):
    return pl.pallas_call(
        _k1_body,
        grid=(NT,),
        in_specs=[
            pl.BlockSpec((T_BLK, D), lambda t: (t, 0)),
            pl.BlockSpec((E, D), lambda t: (0, 0)),
        ],
        out_specs=[
            pl.BlockSpec((K, T_BLK, 1), lambda t: (0, t, 0)),
            pl.BlockSpec((K, T_BLK, 1), lambda t: (0, t, 0)),
            pl.BlockSpec((K, T_BLK, 16), lambda t: (0, t, 0)),
            pl.BlockSpec((1, E), lambda t: (0, 0)),
            pl.BlockSpec((T_BLK, D2), lambda t: (t, 0)),
        ],
        out_shape=[
            jax.ShapeDtypeStruct((K, N, 1), jnp.int32),
            jax.ShapeDtypeStruct((K, N, 1), jnp.int32),
            jax.ShapeDtypeStruct((K, N, 16), jnp.float32),
            jax.ShapeDtypeStruct((1, E), jnp.float32),
            jax.ShapeDtypeStruct((N, D2), jnp.int32),
        ],
        scratch_shapes=[pltpu.VMEM((1, E), jnp.float32)],
    )(x, route_W)


def _k2_body(counts_ref, e_ref, rank_ref, pos_ref, be_ref):
    counts = counts_ref[...].astype(jnp.int32)
    nblk = (counts + (P_BLK - 1)) >> 8
    r8 = lax.broadcasted_iota(jnp.int32, (E, E), 0)
    c8 = lax.broadcasted_iota(jnp.int32, (E, E), 1)
    u_excl = (r8 < c8).astype(jnp.float32)
    u_incl = (r8 <= c8).astype(jnp.float32)
    nblk_f = nblk.astype(jnp.float32)
    off_blocks = lax.dot_general(nblk_f, u_excl, (((1,), (0,)), ((), ())),
                                 preferred_element_type=jnp.float32)
    cum_incl = lax.dot_general(nblk_f, u_incl, (((1,), (0,)), ((), ())),
                               preferred_element_type=jnp.float32)
    padded_off = off_blocks * float(P_BLK)

    e_blk = e_ref[0]
    rank = rank_ref[0]
    lane8 = lax.broadcasted_iota(jnp.int32, (T_BLK, E), 1)
    oh = (e_blk == lane8).astype(jnp.float32)
    base = jnp.sum(oh * padded_off, axis=1, keepdims=True)
    pos_ref[0] = base.astype(jnp.int32) + rank

    b_iota = lax.broadcasted_iota(jnp.int32, (64, E), 0).astype(jnp.float32)
    be = jnp.sum((b_iota >= cum_incl).astype(jnp.float32),
                 axis=1, keepdims=True).astype(jnp.int32)
    be_ref[...] = jnp.minimum(be, E - 1)


def _k2(counts, e_arr, rank_arr):
    return pl.pallas_call(
        _k2_body,
        grid=(K, NT),
        in_specs=[
            pl.BlockSpec((1, E), lambda k, t: (0, 0)),
            pl.BlockSpec((1, T_BLK, 1), lambda k, t: (k, t, 0)),
            pl.BlockSpec((1, T_BLK, 1), lambda k, t: (k, t, 0)),
        ],
        out_specs=[
            pl.BlockSpec((1, T_BLK, 1), lambda k, t: (k, t, 0)),
            pl.BlockSpec((64, 1), lambda k, t: (0, 0)),
        ],
        out_shape=[
            jax.ShapeDtypeStruct((K, N, 1), jnp.int32),
            jax.ShapeDtypeStruct((64, 1), jnp.int32),
        ],
    )(counts, e_arr, rank_arr)


def _unpack_bf16(u):
    lo = lax.bitcast_convert_type((u & 0xFFFF).astype(jnp.uint16), jnp.bfloat16)
    hi = lax.bitcast_convert_type((u >> 16).astype(jnp.uint16), jnp.bfloat16)
    return lo, hi


def _pack_bf16(lo, hi):
    ul = lax.bitcast_convert_type(lo, jnp.uint16).astype(jnp.int32)
    uh = lax.bitcast_convert_type(hi, jnp.uint16).astype(jnp.int32)
    return ul | (uh << 16)


def _k4_body(be_ref, xs_ref, w1_ref, b1_ref, w2_ref, b2_ref, y_ref):
    xlo, xhi = _unpack_bf16(xs_ref[...])
    w1b = w1_ref[0].astype(jnp.bfloat16)
    h = lax.dot_general(xlo, w1b[:, :D2], (((1,), (1,)), ((), ())),
                        preferred_element_type=jnp.float32)
    h += lax.dot_general(xhi, w1b[:, D2:], (((1,), (1,)), ((), ())),
                         preferred_element_type=jnp.float32)
    h = jnp.maximum(h + b1_ref[0], 0.0).astype(jnp.bfloat16)
    w2b = w2_ref[0].astype(jnp.bfloat16)
    y = lax.dot_general(h, w2b, (((1,), (1,)), ((), ())),
                        preferred_element_type=jnp.float32)
    y = jnp.maximum(y + b2_ref[0], 0.0).astype(jnp.bfloat16)
    y_ref[...] = _pack_bf16(y[:, :D2], y[:, D2:])


def _k4(be, x_sorted, W1, b1, W2, b2):
    grid_spec = pltpu.PrefetchScalarGridSpec(
        num_scalar_prefetch=1,
        grid=(NB,),
        in_specs=[
            pl.BlockSpec((P_BLK, D2), lambda i, be_ref: (i, 0)),
            pl.BlockSpec((1, D_FF, D), lambda i, be_ref: (be_ref[i], 0, 0)),
            pl.BlockSpec((1, 1, D_FF), lambda i, be_ref: (be_ref[i], 0, 0)),
            pl.BlockSpec((1, D, D_FF), lambda i, be_ref: (be_ref[i], 0, 0)),
            pl.BlockSpec((1, 1, D), lambda i, be_ref: (be_ref[i], 0, 0)),
        ],
        out_specs=pl.BlockSpec((P_BLK, D2), lambda i, be_ref: (i, 0)),
    )
    return pl.pallas_call(
        _k4_body,
        grid_spec=grid_spec,
        out_shape=jax.ShapeDtypeStruct((MAXP, D2), jnp.int32),
    )(be, x_sorted, W1, b1.reshape(E, 1, D_FF), W2, b2.reshape(E, 1, D))


NW = 32
TOK_W = N // NW
C3 = 64
C5 = 16

_SC_MESH = dict(core_axis_name="c", subcore_axis_name="s")


@functools.partial(
    pl.kernel,
    mesh=plsc.VectorSubcoreMesh(**_SC_MESH),
    out_type=jax.ShapeDtypeStruct((MAXP, D // 2), jnp.int32),
    scratch_types=[
        pltpu.VMEM((C3, D // 2), jnp.int32),
        pltpu.VMEM((C3,), jnp.int32),
        pltpu.VMEM((C3,), jnp.int32),
        pltpu.SemaphoreType.DMA,
        pltpu.SemaphoreType.DMA,
    ],
)
def _k3(x_hbm, p_hbm, xs_hbm, xv, i0v, i1v, sem0, sem1):
    wid = lax.axis_index("s") * 2 + lax.axis_index("c")
    for c in range(TOK_W // C3):
        base = wid * TOK_W + c * C3
        pltpu.sync_copy(x_hbm.at[pl.ds(base, C3)], xv)
        pltpu.sync_copy(p_hbm.at[pl.ds(base, C3)], i0v)
        pltpu.sync_copy(p_hbm.at[pl.ds(N + base, C3)], i1v)
        a = pltpu.async_copy(xv, xs_hbm.at[i0v], sem0)
        b = pltpu.async_copy(xv, xs_hbm.at[i1v], sem1)
        a.wait()
        b.wait()


_NC5 = TOK_W // C5


@functools.partial(
    pl.kernel,
    mesh=plsc.VectorSubcoreMesh(**_SC_MESH),
    out_type=jax.ShapeDtypeStruct((N, D), jnp.float32),
    scratch_types=[
        pltpu.VMEM((TOK_W,), jnp.int32),
        pltpu.VMEM((TOK_W,), jnp.int32),
        pltpu.VMEM((TOK_W, 16), jnp.float32),
        pltpu.VMEM((TOK_W, 16), jnp.float32),
        pltpu.VMEM((3, C5, D2), jnp.int32),
        pltpu.VMEM((3, C5, D2), jnp.int32),
        pltpu.VMEM((2, C5, D), jnp.float32),
        pltpu.SemaphoreType.DMA,
        pltpu.SemaphoreType.DMA,
        pltpu.SemaphoreType.DMA,
        pltpu.SemaphoreType.DMA,
        pltpu.SemaphoreType.DMA,
        pltpu.SemaphoreType.DMA,
        pltpu.SemaphoreType.DMA,
        pltpu.SemaphoreType.DMA,
    ],
)
def _k5(y_hbm, p_hbm, w_hbm, out_hbm,
        i0v, i1v, w0v, w1v, y0v, y1v, ov,
        g0a, g0b, g0c, g1a, g1b, g1c, s0, s1):
    wid = lax.axis_index("s") * 2 + lax.axis_index("c")
    base = wid * TOK_W
    pltpu.sync_copy(p_hbm.at[pl.ds(base, TOK_W)], i0v)
    pltpu.sync_copy(p_hbm.at[pl.ds(N + base, TOK_W)], i1v)
    pltpu.sync_copy(w_hbm.at[pl.ds(base, TOK_W)], w0v)
    pltpu.sync_copy(w_hbm.at[pl.ds(N + base, TOK_W)], w1v)
    gsems = ((g0a, g1a), (g0b, g1b), (g0c, g1c))
    ssems = (s0, s1)
    gather_pend = [None, None, None]
    store_pend = [None, None]

    def issue_gathers(c):
        buf = c % 3
        ga, gb = gsems[buf]
        idx0 = i0v[pl.ds(c * C5, C5)]
        idx1 = i1v[pl.ds(c * C5, C5)]
        a = pltpu.async_copy(y_hbm.at[idx0], y0v.at[buf], ga)
        b = pltpu.async_copy(y_hbm.at[idx1], y1v.at[buf], gb)
        gather_pend[buf] = (a, b)

    for c in range(2):
        issue_gathers(c)
    for c in range(_NC5):
        buf = c % 3
        a, b = gather_pend[buf]
        a.wait()
        b.wait()
        if c + 2 < _NC5:
            issue_gathers(c + 2)
        obuf = c % 2
        if store_pend[obuf] is not None:
            store_pend[obuf].wait()

        def body(t, _, c=c, buf=buf, obuf=obuf):
            w0s = w0v[c * C5 + t, :]
            w1s = w1v[c * C5 + t, :]
            for j in range(D2 // 16):
                sl = pl.ds(j * 16, 16)
                slh = pl.ds(D2 + j * 16, 16)
                u0 = y0v[buf, t, sl]
                u1 = y1v[buf, t, sl]
                lo0 = lax.bitcast_convert_type(u0 << 16, jnp.float32)
                lo1 = lax.bitcast_convert_type(u1 << 16, jnp.float32)
                hi0 = lax.bitcast_convert_type(u0 & jnp.int32(-65536), jnp.float32)
                hi1 = lax.bitcast_convert_type(u1 & jnp.int32(-65536), jnp.float32)
                ov[obuf, t, sl] = w0s * lo0 + w1s * lo1
                ov[obuf, t, slh] = w0s * hi0 + w1s * hi1
            return 0

        lax.fori_loop(0, C5, body, 0)
        store_pend[obuf] = pltpu.async_copy(
            ov.at[obuf], out_hbm.at[pl.ds(base + c * C5, C5)], ssems[obuf])
    for sp in store_pend:
        sp.wait()


def kernel(x, route_W, W1, b1, W2, b2):
    e_arr, rank_arr, wb, counts, xb = _k1(x, route_W)
    pos, be = _k2(counts, e_arr, rank_arr)
    p_flat = pos.reshape(K * N)
    wb_flat = wb.reshape(K * N, 16)
    x_sorted = _k3(xb, p_flat)
    y_sorted = _k4(be.reshape(64), x_sorted, W1, b1, W2, b2)
    return _k5(y_sorted, p_flat, wb_flat)


# speedup vs baseline: 3.1327x; 1.0286x over previous
import functools

import jax
import jax.numpy as jnp
from jax import lax
from jax.experimental import pallas as pl
from jax.experimental.pallas import tpu as pltpu
from jax.experimental.pallas import tpu_sc as plsc

D = 1024
E = 8
K = 2
D_FF = 256
N = 4096

D2 = D // 2
T_BLK = 1024
P_BLK = 256
NT = N // T_BLK
NB = 40
MAXP = NB * P_BLK
NEG_INF = -1e30


def _k1_body(x_ref, rw_ref, e_ref, rank_ref, wb_ref, counts_ref, xb_ref, carry):
    t = pl.program_id(0)

    @pl.when(t == 0)
    def _():
        carry[...] = jnp.zeros_like(carry)

    x = x_ref[...]
    logits = lax.dot_general(x, rw_ref[...], (((1,), (1,)), ((), ())),
                             preferred_element_type=jnp.float32)
    e_iota = lax.broadcasted_iota(jnp.int32, logits.shape, 1)
    m1 = jnp.max(logits, axis=1, keepdims=True)
    i1 = jnp.min(jnp.where(logits == m1, e_iota, E), axis=1, keepdims=True)
    masked = jnp.where(e_iota == i1, NEG_INF, logits)
    m2 = jnp.max(masked, axis=1, keepdims=True)
    i2 = jnp.min(jnp.where(masked == m2, e_iota, E), axis=1, keepdims=True)
    u = jnp.exp(m2 - m1)
    w2 = u / (1.0 + u)
    w1 = 1.0 - w2

    oh0 = (e_iota == i1).astype(jnp.float32)
    oh1 = (e_iota == i2).astype(jnp.float32)
    oh01 = oh0 + oh1
    r_iota = lax.broadcasted_iota(jnp.int32, (T_BLK, T_BLK), 0)
    c_iota = lax.broadcasted_iota(jnp.int32, (T_BLK, T_BLK), 1)
    lstrict = (r_iota > c_iota).astype(jnp.float32)
    within = lax.dot_general(lstrict, oh01, (((1,), (0,)), ((), ())),
                             preferred_element_type=jnp.float32)
    cum0 = carry[...] + within
    cum1 = cum0 + oh0
    rank0 = jnp.sum(cum0 * oh0, axis=1, keepdims=True)
    rank1 = jnp.sum(cum1 * oh1, axis=1, keepdims=True)
    carry[...] += jnp.sum(oh01, axis=0, keepdims=True)

    e_ref[0] = i1
    e_ref[1] = i2
    rank_ref[0] = rank0.astype(jnp.int32)
    rank_ref[1] = rank1.astype(jnp.int32)
    wb_ref[0] = jnp.broadcast_to(w1, (T_BLK, 16))
    wb_ref[1] = jnp.broadcast_to(w2, (T_BLK, 16))
    counts_ref[...] = carry[...]
    xb = x.astype(jnp.bfloat16)
    ul = lax.bitcast_convert_type(xb[:, :D2], jnp.uint16).astype(jnp.int32)
    uh = lax.bitcast_convert_type(xb[:, D2:], jnp.uint16).astype(jnp.int32)
    xb_ref[...] = ul | (uh << 16)


def _k1(x, route_W):
    return pl.pallas_call(
        _k1_body,
        grid=(NT,),
        in_specs=[
            pl.BlockSpec((T_BLK, D), lambda t: (t, 0)),
            pl.BlockSpec((E, D), lambda t: (0, 0)),
        ],
        out_specs=[
            pl.BlockSpec((K, T_BLK, 1), lambda t: (0, t, 0)),
            pl.BlockSpec((K, T_BLK, 1), lambda t: (0, t, 0)),
            pl.BlockSpec((K, T_BLK, 16), lambda t: (0, t, 0)),
            pl.BlockSpec((1, E), lambda t: (0, 0)),
            pl.BlockSpec((T_BLK, D2), lambda t: (t, 0)),
        ],
        out_shape=[
            jax.ShapeDtypeStruct((K, N, 1), jnp.int32),
            jax.ShapeDtypeStruct((K, N, 1), jnp.int32),
            jax.ShapeDtypeStruct((K, N, 16), jnp.float32),
            jax.ShapeDtypeStruct((1, E), jnp.float32),
            jax.ShapeDtypeStruct((N, D2), jnp.int32),
        ],
        scratch_shapes=[pltpu.VMEM((1, E), jnp.float32)],
    )(x, route_W)


def _k2_body(counts_ref, e_ref, rank_ref, pos_ref, be_ref):
    counts = counts_ref[...].astype(jnp.int32)
    nblk = (counts + (P_BLK - 1)) >> 8
    r8 = lax.broadcasted_iota(jnp.int32, (E, E), 0)
    c8 = lax.broadcasted_iota(jnp.int32, (E, E), 1)
    u_excl = (r8 < c8).astype(jnp.float32)
    u_incl = (r8 <= c8).astype(jnp.float32)
    nblk_f = nblk.astype(jnp.float32)
    off_blocks = lax.dot_general(nblk_f, u_excl, (((1,), (0,)), ((), ())),
                                 preferred_element_type=jnp.float32)
    cum_incl = lax.dot_general(nblk_f, u_incl, (((1,), (0,)), ((), ())),
                               preferred_element_type=jnp.float32)
    padded_off = off_blocks * float(P_BLK)

    e_blk = e_ref[0]
    rank = rank_ref[0]
    lane8 = lax.broadcasted_iota(jnp.int32, (T_BLK, E), 1)
    oh = (e_blk == lane8).astype(jnp.float32)
    base = jnp.sum(oh * padded_off, axis=1, keepdims=True)
    pos_ref[0] = base.astype(jnp.int32) + rank

    b_iota = lax.broadcasted_iota(jnp.int32, (64, E), 0).astype(jnp.float32)
    be = jnp.sum((b_iota >= cum_incl).astype(jnp.float32),
                 axis=1, keepdims=True).astype(jnp.int32)
    be_ref[...] = jnp.minimum(be, E - 1)


def _k2(counts, e_arr, rank_arr):
    return pl.pallas_call(
        _k2_body,
        grid=(K, NT),
        in_specs=[
            pl.BlockSpec((1, E), lambda k, t: (0, 0)),
            pl.BlockSpec((1, T_BLK, 1), lambda k, t: (k, t, 0)),
            pl.BlockSpec((1, T_BLK, 1), lambda k, t: (k, t, 0)),
        ],
        out_specs=[
            pl.BlockSpec((1, T_BLK, 1), lambda k, t: (k, t, 0)),
            pl.BlockSpec((64, 1), lambda k, t: (0, 0)),
        ],
        out_shape=[
            jax.ShapeDtypeStruct((K, N, 1), jnp.int32),
            jax.ShapeDtypeStruct((64, 1), jnp.int32),
        ],
    )(counts, e_arr, rank_arr)


def _unpack_bf16(u):
    lo = lax.bitcast_convert_type((u & 0xFFFF).astype(jnp.uint16), jnp.bfloat16)
    hi = lax.bitcast_convert_type((u >> 16).astype(jnp.uint16), jnp.bfloat16)
    return lo, hi


def _pack_bf16(lo, hi):
    ul = lax.bitcast_convert_type(lo, jnp.uint16).astype(jnp.int32)
    uh = lax.bitcast_convert_type(hi, jnp.uint16).astype(jnp.int32)
    return ul | (uh << 16)


def _k4_body(be_ref, xs_ref, w1_ref, b1_ref, w2_ref, b2_ref, y_ref):
    xlo, xhi = _unpack_bf16(xs_ref[...])
    w1b = w1_ref[0].astype(jnp.bfloat16)
    h = lax.dot_general(xlo, w1b[:, :D2], (((1,), (1,)), ((), ())),
                        preferred_element_type=jnp.float32)
    h += lax.dot_general(xhi, w1b[:, D2:], (((1,), (1,)), ((), ())),
                         preferred_element_type=jnp.float32)
    h = jnp.maximum(h + b1_ref[0], 0.0).astype(jnp.bfloat16)
    w2b = w2_ref[0].astype(jnp.bfloat16)
    y = lax.dot_general(h, w2b, (((1,), (1,)), ((), ())),
                        preferred_element_type=jnp.float32)
    y = jnp.maximum(y + b2_ref[0], 0.0).astype(jnp.bfloat16)
    y_ref[...] = _pack_bf16(y[:, :D2], y[:, D2:])


def _k4(be, x_sorted, W1, b1, W2, b2):
    grid_spec = pltpu.PrefetchScalarGridSpec(
        num_scalar_prefetch=1,
        grid=(NB,),
        in_specs=[
            pl.BlockSpec((P_BLK, D2), lambda i, be_ref: (i, 0)),
            pl.BlockSpec((1, D_FF, D), lambda i, be_ref: (be_ref[i], 0, 0)),
            pl.BlockSpec((1, 1, D_FF), lambda i, be_ref: (be_ref[i], 0, 0)),
            pl.BlockSpec((1, D, D_FF), lambda i, be_ref: (be_ref[i], 0, 0)),
            pl.BlockSpec((1, 1, D), lambda i, be_ref: (be_ref[i], 0, 0)),
        ],
        out_specs=pl.BlockSpec((P_BLK, D2), lambda i, be_ref: (i, 0)),
    )
    return pl.pallas_call(
        _k4_body,
        grid_spec=grid_spec,
        out_shape=jax.ShapeDtypeStruct((MAXP, D2), jnp.int32),
    )(be, x_sorted, W1, b1.reshape(E, 1, D_FF), W2, b2.reshape(E, 1, D))


NW = 32
TOK_W = N // NW
C3 = 64
C5 = 16

_SC_MESH = dict(core_axis_name="c", subcore_axis_name="s")


@functools.partial(
    pl.kernel,
    mesh=plsc.VectorSubcoreMesh(**_SC_MESH),
    out_type=jax.ShapeDtypeStruct((MAXP, D // 2), jnp.int32),
    scratch_types=[
        pltpu.VMEM((C3, D // 2), jnp.int32),
        pltpu.VMEM((C3,), jnp.int32),
        pltpu.VMEM((C3,), jnp.int32),
        pltpu.SemaphoreType.DMA,
        pltpu.SemaphoreType.DMA,
    ],
)
def _k3(x_hbm, p_hbm, xs_hbm, xv, i0v, i1v, sem0, sem1):
    wid = lax.axis_index("s") * 2 + lax.axis_index("c")
    for c in range(TOK_W // C3):
        base = wid * TOK_W + c * C3
        pltpu.sync_copy(x_hbm.at[pl.ds(base, C3)], xv)
        pltpu.sync_copy(p_hbm.at[pl.ds(base, C3)], i0v)
        pltpu.sync_copy(p_hbm.at[pl.ds(N + base, C3)], i1v)
        a = pltpu.async_copy(xv, xs_hbm.at[i0v], sem0)
        b = pltpu.async_copy(xv, xs_hbm.at[i1v], sem1)
        a.wait()
        b.wait()


_NC5 = TOK_W // C5


@functools.partial(
    pl.kernel,
    mesh=plsc.VectorSubcoreMesh(**_SC_MESH),
    out_type=jax.ShapeDtypeStruct((N, D), jnp.float32),
    scratch_types=[
        pltpu.VMEM((TOK_W,), jnp.int32),
        pltpu.VMEM((TOK_W,), jnp.int32),
        pltpu.VMEM((TOK_W, 16), jnp.float32),
        pltpu.VMEM((TOK_W, 16), jnp.float32),
        pltpu.VMEM((3, C5, D2), jnp.int32),
        pltpu.VMEM((3, C5, D2), jnp.int32),
        pltpu.VMEM((2, C5, D), jnp.float32),
        pltpu.SemaphoreType.DMA,
        pltpu.SemaphoreType.DMA,
        pltpu.SemaphoreType.DMA,
        pltpu.SemaphoreType.DMA,
        pltpu.SemaphoreType.DMA,
        pltpu.SemaphoreType.DMA,
        pltpu.SemaphoreType.DMA,
        pltpu.SemaphoreType.DMA,
    ],
)
def _k5(y_hbm, p_hbm, w_hbm, out_hbm,
        i0v, i1v, w0v, w1v, y0v, y1v, ov,
        g0a, g0b, g0c, g1a, g1b, g1c, s0, s1):
    wid = lax.axis_index("s") * 2 + lax.axis_index("c")
    base = wid * TOK_W
    pltpu.sync_copy(p_hbm.at[pl.ds(base, TOK_W)], i0v)
    pltpu.sync_copy(p_hbm.at[pl.ds(N + base, TOK_W)], i1v)
    pltpu.sync_copy(w_hbm.at[pl.ds(base, TOK_W)], w0v)
    pltpu.sync_copy(w_hbm.at[pl.ds(N + base, TOK_W)], w1v)
    gsems = ((g0a, g1a), (g0b, g1b), (g0c, g1c))
    ssems = (s0, s1)
    gather_pend = [None, None, None]
    store_pend = [None, None]

    def issue_gathers(c):
        buf = c % 3
        ga, gb = gsems[buf]
        idx0 = i0v[pl.ds(c * C5, C5)]
        idx1 = i1v[pl.ds(c * C5, C5)]
        a = pltpu.async_copy(y_hbm.at[idx0], y0v.at[buf], ga)
        b = pltpu.async_copy(y_hbm.at[idx1], y1v.at[buf], gb)
        gather_pend[buf] = (a, b)

    for c in range(2):
        issue_gathers(c)
    for c in range(_NC5):
        buf = c % 3
        a, b = gather_pend[buf]
        a.wait()
        b.wait()
        if c + 2 < _NC5:
            issue_gathers(c + 2)
        obuf = c % 2
        if store_pend[obuf] is not None:
            store_pend[obuf].wait()

        def body(t, _, c=c, buf=buf, obuf=obuf):
            w0s = w0v[c * C5 + t, :]
            w1s = w1v[c * C5 + t, :]
            for j in range(D2 // 16):
                sl = pl.ds(j * 16, 16)
                slh = pl.ds(D2 + j * 16, 16)
                u0 = y0v[buf, t, sl]
                u1 = y1v[buf, t, sl]
                lo0 = lax.bitcast_convert_type(u0 << 16, jnp.float32)
                lo1 = lax.bitcast_convert_type(u1 << 16, jnp.float32)
                hi0 = lax.bitcast_convert_type(u0 & jnp.int32(-65536), jnp.float32)
                hi1 = lax.bitcast_convert_type(u1 & jnp.int32(-65536), jnp.float32)
                ov[obuf, t, sl] = w0s * lo0 + w1s * lo1
                ov[obuf, t, slh] = w0s * hi0 + w1s * hi1
            return 0

        lax.fori_loop(0, C5, body, 0)
        store_pend[obuf] = pltpu.async_copy(
            ov.at[obuf], out_hbm.at[pl.ds(base + c * C5, C5)], ssems[obuf])
    for sp in store_pend:
        sp.wait()


def kernel(x, route_W, W1, b1, W2, b2):
    e_arr, rank_arr, wb, counts, xb = _k1(x, route_W)
    pos, be = _k2(counts, e_arr, rank_arr)
    p_flat = pos.reshape(K * N)
    wb_flat = wb.reshape(K * N, 16)
    x_sorted = _k3(xb, p_flat)
    y_sorted = _k4(be.reshape(64), x_sorted, W1, b1, W2, b2)
    return _k5(y_sorted, p_flat, wb_flat)


# speedup vs baseline: 3.2763x; 1.0458x over previous
import functools

import jax
import jax.numpy as jnp
from jax import lax
from jax.experimental import pallas as pl
from jax.experimental.pallas import tpu as pltpu
from jax.experimental.pallas import tpu_sc as plsc

D = 1024
E = 8
K = 2
D_FF = 256
N = 4096

D2 = D // 2
T_BLK = 1024
P_BLK = 512
NT = N // T_BLK
NB = 24
MAXP = NB * P_BLK
NEG_INF = -1e30


def _k1_body(x_ref, rw_ref, e_ref, rank_ref, wb_ref, counts_ref, xb_ref, carry):
    t = pl.program_id(0)

    @pl.when(t == 0)
    def _():
        carry[...] = jnp.zeros_like(carry)

    x = x_ref[...]
    logits = lax.dot_general(x, rw_ref[...], (((1,), (1,)), ((), ())),
                             preferred_element_type=jnp.float32)
    e_iota = lax.broadcasted_iota(jnp.int32, logits.shape, 1)
    m1 = jnp.max(logits, axis=1, keepdims=True)
    i1 = jnp.min(jnp.where(logits == m1, e_iota, E), axis=1, keepdims=True)
    masked = jnp.where(e_iota == i1, NEG_INF, logits)
    m2 = jnp.max(masked, axis=1, keepdims=True)
    i2 = jnp.min(jnp.where(masked == m2, e_iota, E), axis=1, keepdims=True)
    u = jnp.exp(m2 - m1)
    w2 = u / (1.0 + u)
    w1 = 1.0 - w2

    oh0 = (e_iota == i1).astype(jnp.float32)
    oh1 = (e_iota == i2).astype(jnp.float32)
    oh01 = oh0 + oh1
    r_iota = lax.broadcasted_iota(jnp.int32, (T_BLK, T_BLK), 0)
    c_iota = lax.broadcasted_iota(jnp.int32, (T_BLK, T_BLK), 1)
    lstrict = (r_iota > c_iota).astype(jnp.float32)
    within = lax.dot_general(lstrict, oh01, (((1,), (0,)), ((), ())),
                             preferred_element_type=jnp.float32)
    cum0 = carry[...] + within
    cum1 = cum0 + oh0
    rank0 = jnp.sum(cum0 * oh0, axis=1, keepdims=True)
    rank1 = jnp.sum(cum1 * oh1, axis=1, keepdims=True)
    carry[...] += jnp.sum(oh01, axis=0, keepdims=True)

    e_ref[0] = i1
    e_ref[1] = i2
    rank_ref[0] = rank0.astype(jnp.int32)
    rank_ref[1] = rank1.astype(jnp.int32)
    wb_ref[0] = jnp.broadcast_to(w1, (T_BLK, 16))
    wb_ref[1] = jnp.broadcast_to(w2, (T_BLK, 16))
    counts_ref[...] = carry[...]
    xb = x.astype(jnp.bfloat16)
    ul = lax.bitcast_convert_type(xb[:, :D2], jnp.uint16).astype(jnp.int32)
    uh = lax.bitcast_convert_type(xb[:, D2:], jnp.uint16).astype(jnp.int32)
    xb_ref[...] = ul | (uh << 16)


def _k1(x, route_W):
    return pl.pallas_call(
        _k1_body,
        grid=(NT,),
        in_specs=[
            pl.BlockSpec((T_BLK, D), lambda t: (t, 0)),
            pl.BlockSpec((E, D), lambda t: (0, 0)),
        ],
        out_specs=[
            pl.BlockSpec((K, T_BLK, 1), lambda t: (0, t, 0)),
            pl.BlockSpec((K, T_BLK, 1), lambda t: (0, t, 0)),
            pl.BlockSpec((K, T_BLK, 16), lambda t: (0, t, 0)),
            pl.BlockSpec((1, E), lambda t: (0, 0)),
            pl.BlockSpec((T_BLK, D2), lambda t: (t, 0)),
        ],
        out_shape=[
            jax.ShapeDtypeStruct((K, N, 1), jnp.int32),
            jax.ShapeDtypeStruct((K, N, 1), jnp.int32),
            jax.ShapeDtypeStruct((K, N, 16), jnp.float32),
            jax.ShapeDtypeStruct((1, E), jnp.float32),
            jax.ShapeDtypeStruct((N, D2), jnp.int32),
        ],
        scratch_shapes=[pltpu.VMEM((1, E), jnp.float32)],
    )(x, route_W)


def _k2_body(counts_ref, e_ref, rank_ref, pos_ref, be_ref):
    counts = counts_ref[...].astype(jnp.int32)
    nblk = (counts + (P_BLK - 1)) >> (P_BLK.bit_length() - 1)
    r8 = lax.broadcasted_iota(jnp.int32, (E, E), 0)
    c8 = lax.broadcasted_iota(jnp.int32, (E, E), 1)
    u_excl = (r8 < c8).astype(jnp.float32)
    u_incl = (r8 <= c8).astype(jnp.float32)
    nblk_f = nblk.astype(jnp.float32)
    off_blocks = lax.dot_general(nblk_f, u_excl, (((1,), (0,)), ((), ())),
                                 preferred_element_type=jnp.float32)
    cum_incl = lax.dot_general(nblk_f, u_incl, (((1,), (0,)), ((), ())),
                               preferred_element_type=jnp.float32)
    padded_off = off_blocks * float(P_BLK)

    e_blk = e_ref[0]
    rank = rank_ref[0]
    lane8 = lax.broadcasted_iota(jnp.int32, (T_BLK, E), 1)
    oh = (e_blk == lane8).astype(jnp.float32)
    base = jnp.sum(oh * padded_off, axis=1, keepdims=True)
    pos_ref[0] = base.astype(jnp.int32) + rank

    b_iota = lax.broadcasted_iota(jnp.int32, (64, E), 0).astype(jnp.float32)
    be = jnp.sum((b_iota >= cum_incl).astype(jnp.float32),
                 axis=1, keepdims=True).astype(jnp.int32)
    be_ref[...] = jnp.minimum(be, E - 1)


def _k2(counts, e_arr, rank_arr):
    return pl.pallas_call(
        _k2_body,
        grid=(K, NT),
        in_specs=[
            pl.BlockSpec((1, E), lambda k, t: (0, 0)),
            pl.BlockSpec((1, T_BLK, 1), lambda k, t: (k, t, 0)),
            pl.BlockSpec((1, T_BLK, 1), lambda k, t: (k, t, 0)),
        ],
        out_specs=[
            pl.BlockSpec((1, T_BLK, 1), lambda k, t: (k, t, 0)),
            pl.BlockSpec((64, 1), lambda k, t: (0, 0)),
        ],
        out_shape=[
            jax.ShapeDtypeStruct((K, N, 1), jnp.int32),
            jax.ShapeDtypeStruct((64, 1), jnp.int32),
        ],
    )(counts, e_arr, rank_arr)


def _unpack_bf16(u):
    lo = lax.bitcast_convert_type((u & 0xFFFF).astype(jnp.uint16), jnp.bfloat16)
    hi = lax.bitcast_convert_type((u >> 16).astype(jnp.uint16), jnp.bfloat16)
    return lo, hi


def _pack_bf16(lo, hi):
    ul = lax.bitcast_convert_type(lo, jnp.uint16).astype(jnp.int32)
    uh = lax.bitcast_convert_type(hi, jnp.uint16).astype(jnp.int32)
    return ul | (uh << 16)


def _k4_body(be_ref, xs_ref, w1_ref, b1_ref, w2_ref, b2_ref, y_ref):
    xlo, xhi = _unpack_bf16(xs_ref[...])
    w1b = w1_ref[0].astype(jnp.bfloat16)
    h = lax.dot_general(xlo, w1b[:, :D2], (((1,), (1,)), ((), ())),
                        preferred_element_type=jnp.float32)
    h += lax.dot_general(xhi, w1b[:, D2:], (((1,), (1,)), ((), ())),
                         preferred_element_type=jnp.float32)
    h = jnp.maximum(h + b1_ref[0], 0.0).astype(jnp.bfloat16)
    w2b = w2_ref[0].astype(jnp.bfloat16)
    y = lax.dot_general(h, w2b, (((1,), (1,)), ((), ())),
                        preferred_element_type=jnp.float32)
    y = jnp.maximum(y + b2_ref[0], 0.0).astype(jnp.bfloat16)
    y_ref[...] = _pack_bf16(y[:, :D2], y[:, D2:])


def _k4(be, x_sorted, W1, b1, W2, b2):
    grid_spec = pltpu.PrefetchScalarGridSpec(
        num_scalar_prefetch=1,
        grid=(NB,),
        in_specs=[
            pl.BlockSpec((P_BLK, D2), lambda i, be_ref: (i, 0)),
            pl.BlockSpec((1, D_FF, D), lambda i, be_ref: (be_ref[i], 0, 0)),
            pl.BlockSpec((1, 1, D_FF), lambda i, be_ref: (be_ref[i], 0, 0)),
            pl.BlockSpec((1, D, D_FF), lambda i, be_ref: (be_ref[i], 0, 0)),
            pl.BlockSpec((1, 1, D), lambda i, be_ref: (be_ref[i], 0, 0)),
        ],
        out_specs=pl.BlockSpec((P_BLK, D2), lambda i, be_ref: (i, 0)),
    )
    return pl.pallas_call(
        _k4_body,
        grid_spec=grid_spec,
        out_shape=jax.ShapeDtypeStruct((MAXP, D2), jnp.int32),
    )(be, x_sorted, W1, b1.reshape(E, 1, D_FF), W2, b2.reshape(E, 1, D))


NW = 32
TOK_W = N // NW
C3 = 64
C5 = 16

_SC_MESH = dict(core_axis_name="c", subcore_axis_name="s")


@functools.partial(
    pl.kernel,
    mesh=plsc.VectorSubcoreMesh(**_SC_MESH),
    out_type=jax.ShapeDtypeStruct((MAXP, D // 2), jnp.int32),
    scratch_types=[
        pltpu.VMEM((C3, D // 2), jnp.int32),
        pltpu.VMEM((C3,), jnp.int32),
        pltpu.VMEM((C3,), jnp.int32),
        pltpu.SemaphoreType.DMA,
        pltpu.SemaphoreType.DMA,
    ],
)
def _k3(x_hbm, p_hbm, xs_hbm, xv, i0v, i1v, sem0, sem1):
    wid = lax.axis_index("s") * 2 + lax.axis_index("c")
    for c in range(TOK_W // C3):
        base = wid * TOK_W + c * C3
        pltpu.sync_copy(x_hbm.at[pl.ds(base, C3)], xv)
        pltpu.sync_copy(p_hbm.at[pl.ds(base, C3)], i0v)
        pltpu.sync_copy(p_hbm.at[pl.ds(N + base, C3)], i1v)
        a = pltpu.async_copy(xv, xs_hbm.at[i0v], sem0)
        b = pltpu.async_copy(xv, xs_hbm.at[i1v], sem1)
        a.wait()
        b.wait()


_NC5 = TOK_W // C5


@functools.partial(
    pl.kernel,
    mesh=plsc.VectorSubcoreMesh(**_SC_MESH),
    out_type=jax.ShapeDtypeStruct((N, D), jnp.float32),
    scratch_types=[
        pltpu.VMEM((TOK_W,), jnp.int32),
        pltpu.VMEM((TOK_W,), jnp.int32),
        pltpu.VMEM((TOK_W, 16), jnp.float32),
        pltpu.VMEM((TOK_W, 16), jnp.float32),
        pltpu.VMEM((3, C5, D2), jnp.int32),
        pltpu.VMEM((3, C5, D2), jnp.int32),
        pltpu.VMEM((2, C5, D), jnp.float32),
        pltpu.SemaphoreType.DMA,
        pltpu.SemaphoreType.DMA,
        pltpu.SemaphoreType.DMA,
        pltpu.SemaphoreType.DMA,
        pltpu.SemaphoreType.DMA,
        pltpu.SemaphoreType.DMA,
        pltpu.SemaphoreType.DMA,
        pltpu.SemaphoreType.DMA,
    ],
)
def _k5(y_hbm, p_hbm, w_hbm, out_hbm,
        i0v, i1v, w0v, w1v, y0v, y1v, ov,
        g0a, g0b, g0c, g1a, g1b, g1c, s0, s1):
    wid = lax.axis_index("s") * 2 + lax.axis_index("c")
    base = wid * TOK_W
    pltpu.sync_copy(p_hbm.at[pl.ds(base, TOK_W)], i0v)
    pltpu.sync_copy(p_hbm.at[pl.ds(N + base, TOK_W)], i1v)
    pltpu.sync_copy(w_hbm.at[pl.ds(base, TOK_W)], w0v)
    pltpu.sync_copy(w_hbm.at[pl.ds(N + base, TOK_W)], w1v)
    gsems = ((g0a, g1a), (g0b, g1b), (g0c, g1c))
    ssems = (s0, s1)
    gather_pend = [None, None, None]
    store_pend = [None, None]

    def issue_gathers(c):
        buf = c % 3
        ga, gb = gsems[buf]
        idx0 = i0v[pl.ds(c * C5, C5)]
        idx1 = i1v[pl.ds(c * C5, C5)]
        a = pltpu.async_copy(y_hbm.at[idx0], y0v.at[buf], ga)
        b = pltpu.async_copy(y_hbm.at[idx1], y1v.at[buf], gb)
        gather_pend[buf] = (a, b)

    for c in range(2):
        issue_gathers(c)
    for c in range(_NC5):
        buf = c % 3
        a, b = gather_pend[buf]
        a.wait()
        b.wait()
        if c + 2 < _NC5:
            issue_gathers(c + 2)
        obuf = c % 2
        if store_pend[obuf] is not None:
            store_pend[obuf].wait()

        def body(t, _, c=c, buf=buf, obuf=obuf):
            w0s = w0v[c * C5 + t, :]
            w1s = w1v[c * C5 + t, :]
            for j in range(D2 // 16):
                sl = pl.ds(j * 16, 16)
                slh = pl.ds(D2 + j * 16, 16)
                u0 = y0v[buf, t, sl]
                u1 = y1v[buf, t, sl]
                lo0 = lax.bitcast_convert_type(u0 << 16, jnp.float32)
                lo1 = lax.bitcast_convert_type(u1 << 16, jnp.float32)
                hi0 = lax.bitcast_convert_type(u0 & jnp.int32(-65536), jnp.float32)
                hi1 = lax.bitcast_convert_type(u1 & jnp.int32(-65536), jnp.float32)
                ov[obuf, t, sl] = w0s * lo0 + w1s * lo1
                ov[obuf, t, slh] = w0s * hi0 + w1s * hi1
            return 0

        lax.fori_loop(0, C5, body, 0)
        store_pend[obuf] = pltpu.async_copy(
            ov.at[obuf], out_hbm.at[pl.ds(base + c * C5, C5)], ssems[obuf])
    for sp in store_pend:
        sp.wait()


def kernel(x, route_W, W1, b1, W2, b2):
    e_arr, rank_arr, wb, counts, xb = _k1(x, route_W)
    pos, be = _k2(counts, e_arr, rank_arr)
    p_flat = pos.reshape(K * N)
    wb_flat = wb.reshape(K * N, 16)
    x_sorted = _k3(xb, p_flat)
    y_sorted = _k4(be.reshape(64), x_sorted, W1, b1, W2, b2)
    return _k5(y_sorted, p_flat, wb_flat)
